# Initial kernel scaffold; baseline (speedup 1.0000x reference)
import jax
import jax.numpy as jnp
from jax.experimental import pallas as pl


def kernel(x, node_batch, edge_index, edge_attr, Wf, bf, Ws, bs, W1, b1, bn_w, bn_b, W2, b2, ln_w, ln_b):
    raise NotImplementedError("write your pallas kernel here")


# speedup vs baseline: 1.8748x; 1.8748x over previous
import functools

import jax
import jax.numpy as jnp
from jax import lax
from jax.experimental import pallas as pl
from jax.experimental.pallas import tpu as pltpu
from jax.experimental.pallas import tpu_sc as plsc

_L = 3
_C = 128
_D = 16
_H = 4 * _C
_N = 10000
_E = 320000
_G = 16
_EPS = 1e-5

_NC = 2
_NS = 16
_NW = _NC * _NS
_EW = _E // _NW
_CH = 80
_NCH = _EW // _CH

_TN = 1000
_TE = 2000


def _sc_mesh():
  return plsc.VectorSubcoreMesh(core_axis_name="c", subcore_axis_name="s")


def _sc_gather(td, ts, dst, src):
  @functools.partial(
      pl.kernel,
      mesh=_sc_mesh(),
      out_type=(
          jax.ShapeDtypeStruct((_E, 2 * _C), jnp.float32),
          jax.ShapeDtypeStruct((_E, 2 * _C), jnp.float32),
      ),
      scratch_types=[
          pltpu.VMEM((_CH,), jnp.int32),
          pltpu.VMEM((_CH,), jnp.int32),
          pltpu.VMEM((_CH, 2 * _C), jnp.float32),
          pltpu.VMEM((_CH, 2 * _C), jnp.float32),
          pltpu.SemaphoreType.DMA,
          pltpu.SemaphoreType.DMA,
      ],
  )
  def k(td_hbm, ts_hbm, dst_hbm, src_hbm, gd_hbm, gs_hbm,
        idxd, idxs, rowsd, rowss, semd, sems):
    wid = lax.axis_index("s") * _NC + lax.axis_index("c")
    base = wid * _EW

    def body(ci, carry):
      off = base + ci * _CH
      pltpu.sync_copy(dst_hbm.at[pl.ds(off, _CH)], idxd)
      pltpu.sync_copy(src_hbm.at[pl.ds(off, _CH)], idxs)
      cpd = pltpu.async_copy(td_hbm.at[idxd], rowsd, semd)
      cps = pltpu.async_copy(ts_hbm.at[idxs], rowss, sems)
      cpd.wait()
      cps.wait()
      pltpu.sync_copy(rowsd, gd_hbm.at[pl.ds(off, _CH)])
      pltpu.sync_copy(rowss, gs_hbm.at[pl.ds(off, _CH)])
      return carry

    lax.fori_loop(0, _NCH, body, 0)

  return k(td, ts, dst, src)


def _sc_scatter(msg, dst, zeros):
  @functools.partial(
      pl.kernel,
      mesh=_sc_mesh(),
      out_type=jax.ShapeDtypeStruct((_NC, _N, _C), jnp.float32),
      scratch_types=[
          pltpu.VMEM((_CH,), jnp.int32),
          pltpu.VMEM((_CH, _C), jnp.float32),
          pltpu.VMEM_SHARED((_N, _C), jnp.float32),
      ],
  )
  def k(msg_hbm, dst_hbm, z_hbm, out_hbm, idx, rows, acc):
    cid = lax.axis_index("c")
    sid = lax.axis_index("s")
    wid = sid * _NC + cid

    @pl.when(sid == 0)
    def _():
      pltpu.sync_copy(z_hbm, acc)

    plsc.subcore_barrier()
    base = wid * _EW

    def body(ci, carry):
      off = base + ci * _CH
      pltpu.sync_copy(dst_hbm.at[pl.ds(off, _CH)], idx)
      pltpu.sync_copy(msg_hbm.at[pl.ds(off, _CH)], rows)
      pltpu.sync_copy(rows, acc.at[idx], add=True)
      return carry

    lax.fori_loop(0, _NCH, body, 0)
    plsc.subcore_barrier()

    @pl.when(sid == 0)
    def _():
      pltpu.sync_copy(acc, out_hbm.at[cid])

  return k(msg, dst, zeros)


def _proj(x, wd, ws):
  def body(x_ref, wd_ref, ws_ref, td_ref, ts_ref):
    xv = x_ref[...]
    td_ref[...] = jnp.dot(xv, wd_ref[...], preferred_element_type=jnp.float32, precision=lax.Precision.HIGHEST)
    ts_ref[...] = jnp.dot(xv, ws_ref[...], preferred_element_type=jnp.float32, precision=lax.Precision.HIGHEST)

  return pl.pallas_call(
      body,
      grid=(_N // _TN,),
      in_specs=[
          pl.BlockSpec((_TN, _C), lambda i: (i, 0)),
          pl.BlockSpec((_C, 2 * _C), lambda i: (0, 0)),
          pl.BlockSpec((_C, 2 * _C), lambda i: (0, 0)),
      ],
      out_specs=[
          pl.BlockSpec((_TN, 2 * _C), lambda i: (i, 0)),
          pl.BlockSpec((_TN, 2 * _C), lambda i: (i, 0)),
      ],
      out_shape=[jax.ShapeDtypeStruct((_N, 2 * _C), jnp.float32)] * 2,
  )(x, wd, ws)


def _edge(gd, gs, ea, wfe, wse, bf8, bs8):
  def body(gd_ref, gs_ref, ea_ref, wfe_ref, wse_ref, bf_ref, bs_ref, o_ref):
    eav = ea_ref[...]
    ef = jnp.dot(eav, wfe_ref[...], preferred_element_type=jnp.float32, precision=lax.Precision.HIGHEST)
    es = jnp.dot(eav, wse_ref[...], preferred_element_type=jnp.float32, precision=lax.Precision.HIGHEST)
    a = gd_ref[:, :_C] + gs_ref[:, :_C] + ef + bf_ref[0:1, :]
    b = gd_ref[:, _C:] + gs_ref[:, _C:] + es + bs_ref[0:1, :]
    gate = 1.0 / (1.0 + jnp.exp(-a))
    core = jnp.maximum(b, 0.0) + jnp.log1p(jnp.exp(-jnp.abs(b)))
    o_ref[...] = gate * core

  return pl.pallas_call(
      body,
      grid=(_E // _TE,),
      in_specs=[
          pl.BlockSpec((_TE, 2 * _C), lambda i: (i, 0)),
          pl.BlockSpec((_TE, 2 * _C), lambda i: (i, 0)),
          pl.BlockSpec((_TE, _D), lambda i: (i, 0)),
          pl.BlockSpec((_D, _C), lambda i: (0, 0)),
          pl.BlockSpec((_D, _C), lambda i: (0, 0)),
          pl.BlockSpec((8, _C), lambda i: (0, 0)),
          pl.BlockSpec((8, _C), lambda i: (0, 0)),
      ],
      out_specs=pl.BlockSpec((_TE, _C), lambda i: (i, 0)),
      out_shape=jax.ShapeDtypeStruct((_E, _C), jnp.float32),
  )(gd, gs, ea, wfe, wse, bf8, bs8)


def _mlp1(x, p0, p1, w1, b18):
  def body(x_ref, p0_ref, p1_ref, w1_ref, b1_ref, x1_ref, h_ref, s_ref):
    i = pl.program_id(0)
    x1 = x_ref[...] + p0_ref[...] + p1_ref[...]
    x1_ref[...] = x1
    h = jnp.dot(x1, w1_ref[...], preferred_element_type=jnp.float32, precision=lax.Precision.HIGHEST)
    h = h + b1_ref[0:1, :]
    h_ref[...] = h
    upd = jnp.concatenate(
        [
            jnp.sum(h, axis=0, keepdims=True),
            jnp.sum(h * h, axis=0, keepdims=True),
            jnp.zeros((6, _H), jnp.float32),
        ],
        axis=0,
    )

    @pl.when(i == 0)
    def _():
      s_ref[...] = upd

    @pl.when(i > 0)
    def _():
      s_ref[...] += upd

  return pl.pallas_call(
      body,
      grid=(_N // _TN,),
      in_specs=[
          pl.BlockSpec((_TN, _C), lambda i: (i, 0)),
          pl.BlockSpec((_TN, _C), lambda i: (i, 0)),
          pl.BlockSpec((_TN, _C), lambda i: (i, 0)),
          pl.BlockSpec((_C, _H), lambda i: (0, 0)),
          pl.BlockSpec((8, _H), lambda i: (0, 0)),
      ],
      out_specs=[
          pl.BlockSpec((_TN, _C), lambda i: (i, 0)),
          pl.BlockSpec((_TN, _H), lambda i: (i, 0)),
          pl.BlockSpec((8, _H), lambda i: (0, 0)),
      ],
      out_shape=[
          jax.ShapeDtypeStruct((_N, _C), jnp.float32),
          jax.ShapeDtypeStruct((_N, _H), jnp.float32),
          jax.ShapeDtypeStruct((8, _H), jnp.float32),
      ],
  )(x, p0, p1, w1, b18)


def _mlp2(h, x1, stats, onehot, bnw8, bnb8, w2, b28):
  def body(h_ref, x1_ref, s_ref, oh_ref, bnw_ref, bnb_ref, w2_ref, b2_ref,
           x2_ref, seg_ref):
    i = pl.program_id(0)
    mu = s_ref[0:1, :] / _N
    var = s_ref[1:2, :] / _N - mu * mu
    hn = (h_ref[...] - mu) * lax.rsqrt(var + _EPS)
    hn = hn * bnw_ref[0:1, :] + bnb_ref[0:1, :]
    hr = jnp.maximum(hn, 0.0)
    xp = jnp.dot(hr, w2_ref[...], preferred_element_type=jnp.float32, precision=lax.Precision.HIGHEST)
    x2 = x1_ref[...] + xp + b2_ref[0:1, :]
    x2_ref[...] = x2
    oh = oh_ref[...]
    dn = (((0,), (0,)), ((), ()))
    s1 = lax.dot_general(oh, x2, dn, preferred_element_type=jnp.float32, precision=lax.Precision.HIGHEST)
    s2 = lax.dot_general(oh, x2 * x2, dn, preferred_element_type=jnp.float32, precision=lax.Precision.HIGHEST)
    dg = lax.dot_general(oh, jnp.ones_like(x2), dn,
                         preferred_element_type=jnp.float32, precision=lax.Precision.HIGHEST)
    upd = jnp.concatenate([s1, s2, dg], axis=0)

    @pl.when(i == 0)
    def _():
      seg_ref[...] = upd

    @pl.when(i > 0)
    def _():
      seg_ref[...] += upd

  return pl.pallas_call(
      body,
      grid=(_N // _TN,),
      in_specs=[
          pl.BlockSpec((_TN, _H), lambda i: (i, 0)),
          pl.BlockSpec((_TN, _C), lambda i: (i, 0)),
          pl.BlockSpec((8, _H), lambda i: (0, 0)),
          pl.BlockSpec((_TN, _G), lambda i: (i, 0)),
          pl.BlockSpec((8, _H), lambda i: (0, 0)),
          pl.BlockSpec((8, _H), lambda i: (0, 0)),
          pl.BlockSpec((_H, _C), lambda i: (0, 0)),
          pl.BlockSpec((8, _C), lambda i: (0, 0)),
      ],
      out_specs=[
          pl.BlockSpec((_TN, _C), lambda i: (i, 0)),
          pl.BlockSpec((3 * _G, _C), lambda i: (0, 0)),
      ],
      out_shape=[
          jax.ShapeDtypeStruct((_N, _C), jnp.float32),
          jax.ShapeDtypeStruct((3 * _G, _C), jnp.float32),
      ],
  )(h, x1, stats, onehot, bnw8, bnb8, w2, b28)


def _gln(x2, onehot, seg, lnw8, lnb8):
  def body(x2_ref, oh_ref, seg_ref, lnw_ref, lnb_ref, o_ref):
    s1 = seg_ref[0:_G, :]
    s2 = seg_ref[_G:2 * _G, :]
    deg = seg_ref[2 * _G:3 * _G, 0:1]
    norm = jnp.maximum(deg, 1.0) * _C
    mean_g = jnp.sum(s1, axis=1, keepdims=True) / norm
    var_g = jnp.sum(s2, axis=1, keepdims=True) / norm - mean_g * mean_g
    inv_g = lax.rsqrt(var_g + _EPS)
    mean_b = jnp.broadcast_to(mean_g, (_G, _C))
    inv_b = jnp.broadcast_to(inv_g, (_G, _C))
    oh = oh_ref[...]
    m = jnp.dot(oh, mean_b, preferred_element_type=jnp.float32, precision=lax.Precision.HIGHEST)
    iv = jnp.dot(oh, inv_b, preferred_element_type=jnp.float32, precision=lax.Precision.HIGHEST)
    o_ref[...] = (x2_ref[...] - m) * iv * lnw_ref[0:1, :] + lnb_ref[0:1, :]

  return pl.pallas_call(
      body,
      grid=(_N // _TN,),
      in_specs=[
          pl.BlockSpec((_TN, _C), lambda i: (i, 0)),
          pl.BlockSpec((_TN, _G), lambda i: (i, 0)),
          pl.BlockSpec((3 * _G, _C), lambda i: (0, 0)),
          pl.BlockSpec((8, _C), lambda i: (0, 0)),
          pl.BlockSpec((8, _C), lambda i: (0, 0)),
      ],
      out_specs=pl.BlockSpec((_TN, _C), lambda i: (i, 0)),
      out_shape=jax.ShapeDtypeStruct((_N, _C), jnp.float32),
  )(x2, onehot, seg, lnw8, lnb8)


def _r8(v):
  return jnp.tile(v.reshape(1, -1), (8, 1))


def kernel(x, node_batch, edge_index, edge_attr, Wf, bf, Ws, bs, W1, b1,
           bn_w, bn_b, W2, b2, ln_w, ln_b):
  src = edge_index[0].astype(jnp.int32)
  dst = edge_index[1].astype(jnp.int32)
  nb = node_batch.astype(jnp.int32)
  onehot = (nb[:, None] == jnp.arange(_G, dtype=jnp.int32)[None, :])
  onehot = onehot.astype(jnp.float32)
  zeros = jnp.zeros((_N, _C), jnp.float32)

  for l in range(_L):
    wd = jnp.concatenate([Wf[l, :_C], Ws[l, :_C]], axis=1)
    wsr = jnp.concatenate([Wf[l, _C:2 * _C], Ws[l, _C:2 * _C]], axis=1)
    wfe = Wf[l, 2 * _C:]
    wse = Ws[l, 2 * _C:]

    td, ts = _proj(x, wd, wsr)
    gd, gs = _sc_gather(td, ts, dst, src)
    msg = _edge(gd, gs, edge_attr, wfe, wse, _r8(bf[l]), _r8(bs[l]))
    parts = _sc_scatter(msg, dst, zeros)
    x1, h, stats = _mlp1(x, parts[0], parts[1], W1[l], _r8(b1[l]))
    x2, seg = _mlp2(h, x1, stats, onehot, _r8(bn_w[l]), _r8(bn_b[l]),
                    W2[l], _r8(b2[l]))
    x = _gln(x2, onehot, seg, _r8(ln_w[l]), _r8(ln_b[l]))
  return x


# speedup vs baseline: 2.2509x; 1.2006x over previous
import functools

import jax
import jax.numpy as jnp
from jax import lax
from jax.experimental import pallas as pl
from jax.experimental.pallas import tpu as pltpu
from jax.experimental.pallas import tpu_sc as plsc

_L = 3
_C = 128
_D = 16
_H = 4 * _C
_N = 10000
_E = 320000
_G = 16
_EPS = 1e-5

_NC = 2
_NS = 16
_NW = _NC * _NS
_EW = _E // _NW
_CH = 80
_NCH = _EW // _CH

_TN = 1000
_TE = 2000


def _sc_mesh():
  return plsc.VectorSubcoreMesh(core_axis_name="c", subcore_axis_name="s")


def _sc_gather(td, ts, dst, src):
  @functools.partial(
      pl.kernel,
      mesh=_sc_mesh(),
      out_type=(
          jax.ShapeDtypeStruct((_E, _C), jnp.int32),
          jax.ShapeDtypeStruct((_E, _C), jnp.int32),
      ),
      scratch_types=[
          pltpu.VMEM((_CH,), jnp.int32),
          pltpu.VMEM((_CH,), jnp.int32),
          pltpu.VMEM((_CH, _C), jnp.int32),
          pltpu.VMEM((_CH, _C), jnp.int32),
          pltpu.SemaphoreType.DMA,
          pltpu.SemaphoreType.DMA,
      ],
  )
  def k(td_hbm, ts_hbm, dst_hbm, src_hbm, gd_hbm, gs_hbm,
        idxd, idxs, rowsd, rowss, semd, sems):
    wid = lax.axis_index("s") * _NC + lax.axis_index("c")
    base = wid * _EW

    def body(ci, carry):
      off = base + ci * _CH
      pltpu.sync_copy(dst_hbm.at[pl.ds(off, _CH)], idxd)
      pltpu.sync_copy(src_hbm.at[pl.ds(off, _CH)], idxs)
      cpd = pltpu.async_copy(td_hbm.at[idxd], rowsd, semd)
      cps = pltpu.async_copy(ts_hbm.at[idxs], rowss, sems)
      cpd.wait()
      cps.wait()
      pltpu.sync_copy(rowsd, gd_hbm.at[pl.ds(off, _CH)])
      pltpu.sync_copy(rowss, gs_hbm.at[pl.ds(off, _CH)])
      return carry

    lax.fori_loop(0, _NCH, body, 0)

  return k(td, ts, dst, src)


def _sc_scatter(msg, dst, zeros):
  @functools.partial(
      pl.kernel,
      mesh=_sc_mesh(),
      out_type=jax.ShapeDtypeStruct((_NC, _N, _C), jnp.float32),
      scratch_types=[
          pltpu.VMEM((_CH,), jnp.int32),
          pltpu.VMEM((_CH, _C), jnp.float32),
          pltpu.VMEM_SHARED((_N, _C), jnp.float32),
      ],
  )
  def k(msg_hbm, dst_hbm, z_hbm, out_hbm, idx, rows, acc):
    cid = lax.axis_index("c")
    sid = lax.axis_index("s")
    wid = sid * _NC + cid

    @pl.when(sid == 0)
    def _():
      pltpu.sync_copy(z_hbm, acc)

    plsc.subcore_barrier()
    base = wid * _EW

    def body(ci, carry):
      off = base + ci * _CH
      pltpu.sync_copy(dst_hbm.at[pl.ds(off, _CH)], idx)
      pltpu.sync_copy(msg_hbm.at[pl.ds(off, _CH)], rows)
      pltpu.sync_copy(rows, acc.at[idx], add=True)
      return carry

    lax.fori_loop(0, _NCH, body, 0)
    plsc.subcore_barrier()

    @pl.when(sid == 0)
    def _():
      pltpu.sync_copy(acc, out_hbm.at[cid])

  return k(msg, dst, zeros)


def _rne_bf16_bits(v):
  bits = lax.bitcast_convert_type(v, jnp.int32)
  return (bits + 0x7FFF + ((bits >> 16) & 1)) >> 16


def _pack2(f, s):
  return (_rne_bf16_bits(s) << 16) | (_rne_bf16_bits(f) & 0xFFFF)


def _unpack_lo(w):
  return lax.bitcast_convert_type(w << 16, jnp.float32)


def _unpack_hi(w):
  return lax.bitcast_convert_type(w & jnp.int32(-65536), jnp.float32)


def _proj(x, wd, ws):
  def body(x_ref, wd_ref, ws_ref, td_ref, ts_ref):
    xv = x_ref[...]
    pd = jnp.dot(xv, wd_ref[...], preferred_element_type=jnp.float32, precision=lax.Precision.HIGHEST)
    ps = jnp.dot(xv, ws_ref[...], preferred_element_type=jnp.float32, precision=lax.Precision.HIGHEST)
    td_ref[...] = _pack2(pd[:, :_C], pd[:, _C:])
    ts_ref[...] = _pack2(ps[:, :_C], ps[:, _C:])

  return pl.pallas_call(
      body,
      grid=(_N // _TN,),
      in_specs=[
          pl.BlockSpec((_TN, _C), lambda i: (i, 0)),
          pl.BlockSpec((_C, 2 * _C), lambda i: (0, 0)),
          pl.BlockSpec((_C, 2 * _C), lambda i: (0, 0)),
      ],
      out_specs=[
          pl.BlockSpec((_TN, _C), lambda i: (i, 0)),
          pl.BlockSpec((_TN, _C), lambda i: (i, 0)),
      ],
      out_shape=[jax.ShapeDtypeStruct((_N, _C), jnp.int32)] * 2,
  )(x, wd, ws)


def _edge(gd, gs, ea, wfe, wse, bf8, bs8):
  def body(gd_ref, gs_ref, ea_ref, wfe_ref, wse_ref, bf_ref, bs_ref, o_ref):
    eav = ea_ref[...]
    ef = jnp.dot(eav, wfe_ref[...], preferred_element_type=jnp.float32, precision=lax.Precision.HIGHEST)
    es = jnp.dot(eav, wse_ref[...], preferred_element_type=jnp.float32, precision=lax.Precision.HIGHEST)
    gd = gd_ref[...]
    gs = gs_ref[...]
    a = _unpack_lo(gd) + _unpack_lo(gs) + ef + bf_ref[0:1, :]
    b = _unpack_hi(gd) + _unpack_hi(gs) + es + bs_ref[0:1, :]
    gate = 1.0 / (1.0 + jnp.exp(-a))
    core = jnp.maximum(b, 0.0) + jnp.log1p(jnp.exp(-jnp.abs(b)))
    o_ref[...] = gate * core

  return pl.pallas_call(
      body,
      grid=(_E // _TE,),
      in_specs=[
          pl.BlockSpec((_TE, _C), lambda i: (i, 0)),
          pl.BlockSpec((_TE, _C), lambda i: (i, 0)),
          pl.BlockSpec((_TE, _D), lambda i: (i, 0)),
          pl.BlockSpec((_D, _C), lambda i: (0, 0)),
          pl.BlockSpec((_D, _C), lambda i: (0, 0)),
          pl.BlockSpec((8, _C), lambda i: (0, 0)),
          pl.BlockSpec((8, _C), lambda i: (0, 0)),
      ],
      out_specs=pl.BlockSpec((_TE, _C), lambda i: (i, 0)),
      out_shape=jax.ShapeDtypeStruct((_E, _C), jnp.float32),
  )(gd, gs, ea, wfe, wse, bf8, bs8)


def _mlp1(x, p0, p1, w1, b18):
  def body(x_ref, p0_ref, p1_ref, w1_ref, b1_ref, x1_ref, h_ref, s_ref):
    i = pl.program_id(0)
    x1 = x_ref[...] + p0_ref[...] + p1_ref[...]
    x1_ref[...] = x1
    h = jnp.dot(x1, w1_ref[...], preferred_element_type=jnp.float32, precision=lax.Precision.HIGHEST)
    h = h + b1_ref[0:1, :]
    h_ref[...] = h
    upd = jnp.concatenate(
        [
            jnp.sum(h, axis=0, keepdims=True),
            jnp.sum(h * h, axis=0, keepdims=True),
            jnp.zeros((6, _H), jnp.float32),
        ],
        axis=0,
    )

    @pl.when(i == 0)
    def _():
      s_ref[...] = upd

    @pl.when(i > 0)
    def _():
      s_ref[...] += upd

  return pl.pallas_call(
      body,
      grid=(_N // _TN,),
      in_specs=[
          pl.BlockSpec((_TN, _C), lambda i: (i, 0)),
          pl.BlockSpec((_TN, _C), lambda i: (i, 0)),
          pl.BlockSpec((_TN, _C), lambda i: (i, 0)),
          pl.BlockSpec((_C, _H), lambda i: (0, 0)),
          pl.BlockSpec((8, _H), lambda i: (0, 0)),
      ],
      out_specs=[
          pl.BlockSpec((_TN, _C), lambda i: (i, 0)),
          pl.BlockSpec((_TN, _H), lambda i: (i, 0)),
          pl.BlockSpec((8, _H), lambda i: (0, 0)),
      ],
      out_shape=[
          jax.ShapeDtypeStruct((_N, _C), jnp.float32),
          jax.ShapeDtypeStruct((_N, _H), jnp.float32),
          jax.ShapeDtypeStruct((8, _H), jnp.float32),
      ],
  )(x, p0, p1, w1, b18)


def _mlp2(h, x1, stats, onehot, bnw8, bnb8, w2, b28):
  def body(h_ref, x1_ref, s_ref, oh_ref, bnw_ref, bnb_ref, w2_ref, b2_ref,
           x2_ref, seg_ref):
    i = pl.program_id(0)
    mu = s_ref[0:1, :] / _N
    var = s_ref[1:2, :] / _N - mu * mu
    hn = (h_ref[...] - mu) * lax.rsqrt(var + _EPS)
    hn = hn * bnw_ref[0:1, :] + bnb_ref[0:1, :]
    hr = jnp.maximum(hn, 0.0)
    xp = jnp.dot(hr, w2_ref[...], preferred_element_type=jnp.float32, precision=lax.Precision.HIGHEST)
    x2 = x1_ref[...] + xp + b2_ref[0:1, :]
    x2_ref[...] = x2
    oh = oh_ref[...]
    dn = (((0,), (0,)), ((), ()))
    s1 = lax.dot_general(oh, x2, dn, preferred_element_type=jnp.float32, precision=lax.Precision.HIGHEST)
    s2 = lax.dot_general(oh, x2 * x2, dn, preferred_element_type=jnp.float32, precision=lax.Precision.HIGHEST)
    dg = lax.dot_general(oh, jnp.ones_like(x2), dn,
                         preferred_element_type=jnp.float32, precision=lax.Precision.HIGHEST)
    upd = jnp.concatenate([s1, s2, dg], axis=0)

    @pl.when(i == 0)
    def _():
      seg_ref[...] = upd

    @pl.when(i > 0)
    def _():
      seg_ref[...] += upd

  return pl.pallas_call(
      body,
      grid=(_N // _TN,),
      in_specs=[
          pl.BlockSpec((_TN, _H), lambda i: (i, 0)),
          pl.BlockSpec((_TN, _C), lambda i: (i, 0)),
          pl.BlockSpec((8, _H), lambda i: (0, 0)),
          pl.BlockSpec((_TN, _G), lambda i: (i, 0)),
          pl.BlockSpec((8, _H), lambda i: (0, 0)),
          pl.BlockSpec((8, _H), lambda i: (0, 0)),
          pl.BlockSpec((_H, _C), lambda i: (0, 0)),
          pl.BlockSpec((8, _C), lambda i: (0, 0)),
      ],
      out_specs=[
          pl.BlockSpec((_TN, _C), lambda i: (i, 0)),
          pl.BlockSpec((3 * _G, _C), lambda i: (0, 0)),
      ],
      out_shape=[
          jax.ShapeDtypeStruct((_N, _C), jnp.float32),
          jax.ShapeDtypeStruct((3 * _G, _C), jnp.float32),
      ],
  )(h, x1, stats, onehot, bnw8, bnb8, w2, b28)


def _gln(x2, onehot, seg, lnw8, lnb8):
  def body(x2_ref, oh_ref, seg_ref, lnw_ref, lnb_ref, o_ref):
    s1 = seg_ref[0:_G, :]
    s2 = seg_ref[_G:2 * _G, :]
    deg = seg_ref[2 * _G:3 * _G, 0:1]
    norm = jnp.maximum(deg, 1.0) * _C
    mean_g = jnp.sum(s1, axis=1, keepdims=True) / norm
    var_g = jnp.sum(s2, axis=1, keepdims=True) / norm - mean_g * mean_g
    inv_g = lax.rsqrt(var_g + _EPS)
    mean_b = jnp.broadcast_to(mean_g, (_G, _C))
    inv_b = jnp.broadcast_to(inv_g, (_G, _C))
    oh = oh_ref[...]
    m = jnp.dot(oh, mean_b, preferred_element_type=jnp.float32, precision=lax.Precision.HIGHEST)
    iv = jnp.dot(oh, inv_b, preferred_element_type=jnp.float32, precision=lax.Precision.HIGHEST)
    o_ref[...] = (x2_ref[...] - m) * iv * lnw_ref[0:1, :] + lnb_ref[0:1, :]

  return pl.pallas_call(
      body,
      grid=(_N // _TN,),
      in_specs=[
          pl.BlockSpec((_TN, _C), lambda i: (i, 0)),
          pl.BlockSpec((_TN, _G), lambda i: (i, 0)),
          pl.BlockSpec((3 * _G, _C), lambda i: (0, 0)),
          pl.BlockSpec((8, _C), lambda i: (0, 0)),
          pl.BlockSpec((8, _C), lambda i: (0, 0)),
      ],
      out_specs=pl.BlockSpec((_TN, _C), lambda i: (i, 0)),
      out_shape=jax.ShapeDtypeStruct((_N, _C), jnp.float32),
  )(x2, onehot, seg, lnw8, lnb8)


def _r8(v):
  return jnp.tile(v.reshape(1, -1), (8, 1))


def kernel(x, node_batch, edge_index, edge_attr, Wf, bf, Ws, bs, W1, b1,
           bn_w, bn_b, W2, b2, ln_w, ln_b):
  src = edge_index[0].astype(jnp.int32)
  dst = edge_index[1].astype(jnp.int32)
  nb = node_batch.astype(jnp.int32)
  onehot = (nb[:, None] == jnp.arange(_G, dtype=jnp.int32)[None, :])
  onehot = onehot.astype(jnp.float32)
  zeros = jnp.zeros((_N, _C), jnp.float32)

  for l in range(_L):
    wd = jnp.concatenate([Wf[l, :_C], Ws[l, :_C]], axis=1)
    wsr = jnp.concatenate([Wf[l, _C:2 * _C], Ws[l, _C:2 * _C]], axis=1)
    wfe = Wf[l, 2 * _C:]
    wse = Ws[l, 2 * _C:]

    td, ts = _proj(x, wd, wsr)
    gd, gs = _sc_gather(td, ts, dst, src)
    msg = _edge(gd, gs, edge_attr, wfe, wse, _r8(bf[l]), _r8(bs[l]))
    parts = _sc_scatter(msg, dst, zeros)
    x1, h, stats = _mlp1(x, parts[0], parts[1], W1[l], _r8(b1[l]))
    x2, seg = _mlp2(h, x1, stats, onehot, _r8(bn_w[l]), _r8(bn_b[l]),
                    W2[l], _r8(b2[l]))
    x = _gln(x2, onehot, seg, _r8(ln_w[l]), _r8(ln_b[l]))
  return x


# speedup vs baseline: 2.9287x; 1.3011x over previous
import functools

import jax
import jax.numpy as jnp
from jax import lax
from jax.experimental import pallas as pl
from jax.experimental.pallas import tpu as pltpu
from jax.experimental.pallas import tpu_sc as plsc

_L = 3
_C = 128
_D = 16
_H = 4 * _C
_N = 10000
_E = 320000
_G = 16
_EPS = 1e-5

_NC = 2
_NS = 16
_NW = _NC * _NS
_EW = _E // _NW
_CH = 128
_NPAIR = 39
_TAIL = _EW - 2 * _NPAIR * _CH
_TOFF = 2 * _NPAIR * _CH

_TN = 1000
_TE = 2000


def _sc_mesh():
  return plsc.VectorSubcoreMesh(core_axis_name="c", subcore_axis_name="s")


def _sc_gather(td, ts, dst, src):
  @functools.partial(
      pl.kernel,
      mesh=_sc_mesh(),
      out_type=(
          jax.ShapeDtypeStruct((_E, _C), jnp.int32),
          jax.ShapeDtypeStruct((_E, _C), jnp.int32),
      ),
      scratch_types=[
          pltpu.VMEM((_EW,), jnp.int32),
          pltpu.VMEM((_EW,), jnp.int32),
          [pltpu.VMEM((_CH, _C), jnp.int32)] * 2,
          [pltpu.VMEM((_CH, _C), jnp.int32)] * 2,
          pltpu.VMEM((_TAIL, _C), jnp.int32),
          pltpu.VMEM((_TAIL, _C), jnp.int32),
          [pltpu.SemaphoreType.DMA] * 2,
          [pltpu.SemaphoreType.DMA] * 2,
          [pltpu.SemaphoreType.DMA] * 2,
          [pltpu.SemaphoreType.DMA] * 2,
          pltpu.SemaphoreType.DMA,
      ],
  )
  def k(td_hbm, ts_hbm, dst_hbm, src_hbm, gd_hbm, gs_hbm,
        idxd, idxs, rowsd, rowss, taild, tails,
        gsemd, gsems, wsemd, wsems, tsem):
    wid = lax.axis_index("s") * _NC + lax.axis_index("c")
    base = wid * _EW
    pltpu.sync_copy(dst_hbm.at[pl.ds(base, _EW)], idxd)
    pltpu.sync_copy(src_hbm.at[pl.ds(base, _EW)], idxs)

    def body(g, carry):
      gds = []
      for b in range(2):
        c = 2 * g + b
        osl = pl.ds(base + c * _CH, _CH)
        isl = pl.ds(c * _CH, _CH)

        @pl.when(g > 0)
        def _():
          pltpu.make_async_copy(rowsd[b], gd_hbm.at[osl], wsemd[b]).wait()
          pltpu.make_async_copy(rowss[b], gs_hbm.at[osl], wsems[b]).wait()

        gds.append((
            pltpu.async_copy(td_hbm.at[idxd.at[isl]], rowsd[b], gsemd[b]),
            pltpu.async_copy(ts_hbm.at[idxs.at[isl]], rowss[b], gsems[b]),
        ))
      for b in range(2):
        c = 2 * g + b
        osl = pl.ds(base + c * _CH, _CH)
        cpd, cps = gds[b]
        cpd.wait()
        cps.wait()
        pltpu.async_copy(rowsd[b], gd_hbm.at[osl], wsemd[b])
        pltpu.async_copy(rowss[b], gs_hbm.at[osl], wsems[b])
      return carry

    lax.fori_loop(0, _NPAIR, body, 0)

    tsl = pl.ds(base + _TOFF, _TAIL)
    cpd = pltpu.async_copy(td_hbm.at[idxd.at[pl.ds(_TOFF, _TAIL)]], taild, tsem)
    cpd.wait()
    cps = pltpu.async_copy(ts_hbm.at[idxs.at[pl.ds(_TOFF, _TAIL)]], tails, tsem)
    cps.wait()
    pltpu.sync_copy(taild, gd_hbm.at[tsl])
    pltpu.sync_copy(tails, gs_hbm.at[tsl])
    for b in range(2):
      c = 2 * (_NPAIR - 1) + b
      osl = pl.ds(base + c * _CH, _CH)
      pltpu.make_async_copy(rowsd[b], gd_hbm.at[osl], wsemd[b]).wait()
      pltpu.make_async_copy(rowss[b], gs_hbm.at[osl], wsems[b]).wait()

  return k(td, ts, dst, src)


def _sc_scatter(msg, dst, zeros):
  @functools.partial(
      pl.kernel,
      mesh=_sc_mesh(),
      out_type=jax.ShapeDtypeStruct((_NC, _N, _C), jnp.float32),
      scratch_types=[
          [pltpu.VMEM((_CH,), jnp.int32)] * 2,
          [pltpu.VMEM((_CH, _C), jnp.float32)] * 2,
          pltpu.VMEM((_TAIL,), jnp.int32),
          pltpu.VMEM((_TAIL, _C), jnp.float32),
          pltpu.VMEM_SHARED((_N, _C), jnp.float32),
          [pltpu.SemaphoreType.DMA] * 2,
          [pltpu.SemaphoreType.DMA] * 2,
          [pltpu.SemaphoreType.DMA] * 2,
          pltpu.SemaphoreType.DMA,
      ],
  )
  def k(msg_hbm, dst_hbm, z_hbm, out_hbm, idx, rows, idxt, rowst, acc,
        isem, lsem, ssem, tsem):
    cid = lax.axis_index("c")
    sid = lax.axis_index("s")
    wid = sid * _NC + cid

    @pl.when(sid == 0)
    def _():
      pltpu.sync_copy(z_hbm, acc)

    plsc.subcore_barrier()
    base = wid * _EW

    def body(g, carry):
      lds = []
      for b in range(2):
        c = 2 * g + b
        osl = pl.ds(base + c * _CH, _CH)

        @pl.when(g > 0)
        def _():
          pltpu.make_async_copy(rows[b], acc.at[idx[b]], ssem[b]).wait()

        lds.append((
            pltpu.async_copy(dst_hbm.at[osl], idx[b], isem[b]),
            pltpu.async_copy(msg_hbm.at[osl], rows[b], lsem[b]),
        ))
      for b in range(2):
        cpi, cpm = lds[b]
        cpi.wait()
        cpm.wait()
        pltpu.async_copy(rows[b], acc.at[idx[b]], ssem[b], add=True)
      return carry

    lax.fori_loop(0, _NPAIR, body, 0)
    for b in range(2):
      pltpu.make_async_copy(rows[b], acc.at[idx[b]], ssem[b]).wait()
    tsl = pl.ds(base + _TOFF, _TAIL)
    cpi = pltpu.async_copy(dst_hbm.at[tsl], idxt, tsem)
    cpi.wait()
    cpm = pltpu.async_copy(msg_hbm.at[tsl], rowst, tsem)
    cpm.wait()
    pltpu.sync_copy(rowst, acc.at[idxt], add=True)

    plsc.subcore_barrier()

    @pl.when(sid == 0)
    def _():
      pltpu.sync_copy(acc, out_hbm.at[cid])

  return k(msg, dst, zeros)


def _rne_bf16_bits(v):
  bits = lax.bitcast_convert_type(v, jnp.int32)
  return (bits + 0x7FFF + ((bits >> 16) & 1)) >> 16


def _pack2(f, s):
  return (_rne_bf16_bits(s) << 16) | (_rne_bf16_bits(f) & 0xFFFF)


def _unpack_lo(w):
  return lax.bitcast_convert_type(w << 16, jnp.float32)


def _unpack_hi(w):
  return lax.bitcast_convert_type(w & jnp.int32(-65536), jnp.float32)


def _proj(x, wd, ws):
  def body(x_ref, wd_ref, ws_ref, td_ref, ts_ref):
    xv = x_ref[...]
    pd = jnp.dot(xv, wd_ref[...], preferred_element_type=jnp.float32, precision=lax.Precision.HIGHEST)
    ps = jnp.dot(xv, ws_ref[...], preferred_element_type=jnp.float32, precision=lax.Precision.HIGHEST)
    td_ref[...] = _pack2(pd[:, :_C], pd[:, _C:])
    ts_ref[...] = _pack2(ps[:, :_C], ps[:, _C:])

  return pl.pallas_call(
      body,
      grid=(_N // _TN,),
      in_specs=[
          pl.BlockSpec((_TN, _C), lambda i: (i, 0)),
          pl.BlockSpec((_C, 2 * _C), lambda i: (0, 0)),
          pl.BlockSpec((_C, 2 * _C), lambda i: (0, 0)),
      ],
      out_specs=[
          pl.BlockSpec((_TN, _C), lambda i: (i, 0)),
          pl.BlockSpec((_TN, _C), lambda i: (i, 0)),
      ],
      out_shape=[jax.ShapeDtypeStruct((_N, _C), jnp.int32)] * 2,
  )(x, wd, ws)


def _edge(gd, gs, ea, wfe, wse, bf8, bs8):
  def body(gd_ref, gs_ref, ea_ref, wfe_ref, wse_ref, bf_ref, bs_ref, o_ref):
    eav = ea_ref[...]
    ef = jnp.dot(eav, wfe_ref[...], preferred_element_type=jnp.float32, precision=lax.Precision.HIGHEST)
    es = jnp.dot(eav, wse_ref[...], preferred_element_type=jnp.float32, precision=lax.Precision.HIGHEST)
    gd = gd_ref[...]
    gs = gs_ref[...]
    a = _unpack_lo(gd) + _unpack_lo(gs) + ef + bf_ref[0:1, :]
    b = _unpack_hi(gd) + _unpack_hi(gs) + es + bs_ref[0:1, :]
    gate = 1.0 / (1.0 + jnp.exp(-a))
    core = jnp.maximum(b, 0.0) + jnp.log1p(jnp.exp(-jnp.abs(b)))
    o_ref[...] = gate * core

  return pl.pallas_call(
      body,
      grid=(_E // _TE,),
      in_specs=[
          pl.BlockSpec((_TE, _C), lambda i: (i, 0)),
          pl.BlockSpec((_TE, _C), lambda i: (i, 0)),
          pl.BlockSpec((_TE, _D), lambda i: (i, 0)),
          pl.BlockSpec((_D, _C), lambda i: (0, 0)),
          pl.BlockSpec((_D, _C), lambda i: (0, 0)),
          pl.BlockSpec((8, _C), lambda i: (0, 0)),
          pl.BlockSpec((8, _C), lambda i: (0, 0)),
      ],
      out_specs=pl.BlockSpec((_TE, _C), lambda i: (i, 0)),
      out_shape=jax.ShapeDtypeStruct((_E, _C), jnp.float32),
  )(gd, gs, ea, wfe, wse, bf8, bs8)


def _mlp1(x, p0, p1, w1, b18):
  def body(x_ref, p0_ref, p1_ref, w1_ref, b1_ref, x1_ref, h_ref, s_ref):
    i = pl.program_id(0)
    x1 = x_ref[...] + p0_ref[...] + p1_ref[...]
    x1_ref[...] = x1
    h = jnp.dot(x1, w1_ref[...], preferred_element_type=jnp.float32, precision=lax.Precision.HIGHEST)
    h = h + b1_ref[0:1, :]
    h_ref[...] = h
    upd = jnp.concatenate(
        [
            jnp.sum(h, axis=0, keepdims=True),
            jnp.sum(h * h, axis=0, keepdims=True),
            jnp.zeros((6, _H), jnp.float32),
        ],
        axis=0,
    )

    @pl.when(i == 0)
    def _():
      s_ref[...] = upd

    @pl.when(i > 0)
    def _():
      s_ref[...] += upd

  return pl.pallas_call(
      body,
      grid=(_N // _TN,),
      in_specs=[
          pl.BlockSpec((_TN, _C), lambda i: (i, 0)),
          pl.BlockSpec((_TN, _C), lambda i: (i, 0)),
          pl.BlockSpec((_TN, _C), lambda i: (i, 0)),
          pl.BlockSpec((_C, _H), lambda i: (0, 0)),
          pl.BlockSpec((8, _H), lambda i: (0, 0)),
      ],
      out_specs=[
          pl.BlockSpec((_TN, _C), lambda i: (i, 0)),
          pl.BlockSpec((_TN, _H), lambda i: (i, 0)),
          pl.BlockSpec((8, _H), lambda i: (0, 0)),
      ],
      out_shape=[
          jax.ShapeDtypeStruct((_N, _C), jnp.float32),
          jax.ShapeDtypeStruct((_N, _H), jnp.float32),
          jax.ShapeDtypeStruct((8, _H), jnp.float32),
      ],
  )(x, p0, p1, w1, b18)


def _mlp2(h, x1, stats, onehot, bnw8, bnb8, w2, b28):
  def body(h_ref, x1_ref, s_ref, oh_ref, bnw_ref, bnb_ref, w2_ref, b2_ref,
           x2_ref, seg_ref):
    i = pl.program_id(0)
    mu = s_ref[0:1, :] / _N
    var = s_ref[1:2, :] / _N - mu * mu
    hn = (h_ref[...] - mu) * lax.rsqrt(var + _EPS)
    hn = hn * bnw_ref[0:1, :] + bnb_ref[0:1, :]
    hr = jnp.maximum(hn, 0.0)
    xp = jnp.dot(hr, w2_ref[...], preferred_element_type=jnp.float32, precision=lax.Precision.HIGHEST)
    x2 = x1_ref[...] + xp + b2_ref[0:1, :]
    x2_ref[...] = x2
    oh = oh_ref[...]
    dn = (((0,), (0,)), ((), ()))
    s1 = lax.dot_general(oh, x2, dn, preferred_element_type=jnp.float32, precision=lax.Precision.HIGHEST)
    s2 = lax.dot_general(oh, x2 * x2, dn, preferred_element_type=jnp.float32, precision=lax.Precision.HIGHEST)
    dg = lax.dot_general(oh, jnp.ones_like(x2), dn,
                         preferred_element_type=jnp.float32, precision=lax.Precision.HIGHEST)
    upd = jnp.concatenate([s1, s2, dg], axis=0)

    @pl.when(i == 0)
    def _():
      seg_ref[...] = upd

    @pl.when(i > 0)
    def _():
      seg_ref[...] += upd

  return pl.pallas_call(
      body,
      grid=(_N // _TN,),
      in_specs=[
          pl.BlockSpec((_TN, _H), lambda i: (i, 0)),
          pl.BlockSpec((_TN, _C), lambda i: (i, 0)),
          pl.BlockSpec((8, _H), lambda i: (0, 0)),
          pl.BlockSpec((_TN, _G), lambda i: (i, 0)),
          pl.BlockSpec((8, _H), lambda i: (0, 0)),
          pl.BlockSpec((8, _H), lambda i: (0, 0)),
          pl.BlockSpec((_H, _C), lambda i: (0, 0)),
          pl.BlockSpec((8, _C), lambda i: (0, 0)),
      ],
      out_specs=[
          pl.BlockSpec((_TN, _C), lambda i: (i, 0)),
          pl.BlockSpec((3 * _G, _C), lambda i: (0, 0)),
      ],
      out_shape=[
          jax.ShapeDtypeStruct((_N, _C), jnp.float32),
          jax.ShapeDtypeStruct((3 * _G, _C), jnp.float32),
      ],
  )(h, x1, stats, onehot, bnw8, bnb8, w2, b28)


def _gln(x2, onehot, seg, lnw8, lnb8):
  def body(x2_ref, oh_ref, seg_ref, lnw_ref, lnb_ref, o_ref):
    s1 = seg_ref[0:_G, :]
    s2 = seg_ref[_G:2 * _G, :]
    deg = seg_ref[2 * _G:3 * _G, 0:1]
    norm = jnp.maximum(deg, 1.0) * _C
    mean_g = jnp.sum(s1, axis=1, keepdims=True) / norm
    var_g = jnp.sum(s2, axis=1, keepdims=True) / norm - mean_g * mean_g
    inv_g = lax.rsqrt(var_g + _EPS)
    mean_b = jnp.broadcast_to(mean_g, (_G, _C))
    inv_b = jnp.broadcast_to(inv_g, (_G, _C))
    oh = oh_ref[...]
    m = jnp.dot(oh, mean_b, preferred_element_type=jnp.float32, precision=lax.Precision.HIGHEST)
    iv = jnp.dot(oh, inv_b, preferred_element_type=jnp.float32, precision=lax.Precision.HIGHEST)
    o_ref[...] = (x2_ref[...] - m) * iv * lnw_ref[0:1, :] + lnb_ref[0:1, :]

  return pl.pallas_call(
      body,
      grid=(_N // _TN,),
      in_specs=[
          pl.BlockSpec((_TN, _C), lambda i: (i, 0)),
          pl.BlockSpec((_TN, _G), lambda i: (i, 0)),
          pl.BlockSpec((3 * _G, _C), lambda i: (0, 0)),
          pl.BlockSpec((8, _C), lambda i: (0, 0)),
          pl.BlockSpec((8, _C), lambda i: (0, 0)),
      ],
      out_specs=pl.BlockSpec((_TN, _C), lambda i: (i, 0)),
      out_shape=jax.ShapeDtypeStruct((_N, _C), jnp.float32),
  )(x2, onehot, seg, lnw8, lnb8)


def _r8(v):
  return jnp.tile(v.reshape(1, -1), (8, 1))


def kernel(x, node_batch, edge_index, edge_attr, Wf, bf, Ws, bs, W1, b1,
           bn_w, bn_b, W2, b2, ln_w, ln_b):
  src = edge_index[0].astype(jnp.int32)
  dst = edge_index[1].astype(jnp.int32)
  nb = node_batch.astype(jnp.int32)
  onehot = (nb[:, None] == jnp.arange(_G, dtype=jnp.int32)[None, :])
  onehot = onehot.astype(jnp.float32)
  zeros = jnp.zeros((_N, _C), jnp.float32)

  for l in range(_L):
    wd = jnp.concatenate([Wf[l, :_C], Ws[l, :_C]], axis=1)
    wsr = jnp.concatenate([Wf[l, _C:2 * _C], Ws[l, _C:2 * _C]], axis=1)
    wfe = Wf[l, 2 * _C:]
    wse = Ws[l, 2 * _C:]

    td, ts = _proj(x, wd, wsr)
    gd, gs = _sc_gather(td, ts, dst, src)
    msg = _edge(gd, gs, edge_attr, wfe, wse, _r8(bf[l]), _r8(bs[l]))
    parts = _sc_scatter(msg, dst, zeros)
    x1, h, stats = _mlp1(x, parts[0], parts[1], W1[l], _r8(b1[l]))
    x2, seg = _mlp2(h, x1, stats, onehot, _r8(bn_w[l]), _r8(bn_b[l]),
                    W2[l], _r8(b2[l]))
    x = _gln(x2, onehot, seg, _r8(ln_w[l]), _r8(ln_b[l]))
  return x


# speedup vs baseline: 3.2838x; 1.1212x over previous
import functools

import jax
import jax.numpy as jnp
from jax import lax
from jax.experimental import pallas as pl
from jax.experimental.pallas import tpu as pltpu
from jax.experimental.pallas import tpu_sc as plsc

_L = 3
_C = 128
_D = 16
_H = 4 * _C
_N = 10000
_E = 320000
_G = 16
_EPS = 1e-5

_NC = 2
_NS = 16
_NW = _NC * _NS
_EH = _E // 2
_EW = _EH // _NW
_CH = 128
_NFULL = _EW // _CH
_NPAIR = _NFULL // 2
_CREM = 2 * _NPAIR
_TAIL = _EW - _NFULL * _CH
_TOFF = _NFULL * _CH

_TN = 1000
_TE = 2000


def _sc_mesh():
  return plsc.VectorSubcoreMesh(core_axis_name="c", subcore_axis_name="s")


def _sc_gather(td, ts, dst, src):
  @functools.partial(
      pl.kernel,
      mesh=_sc_mesh(),
      out_type=(
          jax.ShapeDtypeStruct((_EH, _C), jnp.int32),
          jax.ShapeDtypeStruct((_EH, _C), jnp.int32),
      ),
      scratch_types=[
          pltpu.VMEM((_EW,), jnp.int32),
          pltpu.VMEM((_EW,), jnp.int32),
          [pltpu.VMEM((_CH, _C), jnp.int32)] * 2,
          [pltpu.VMEM((_CH, _C), jnp.int32)] * 2,
          pltpu.VMEM((_TAIL, _C), jnp.int32),
          pltpu.VMEM((_TAIL, _C), jnp.int32),
          [pltpu.SemaphoreType.DMA] * 2,
          [pltpu.SemaphoreType.DMA] * 2,
          [pltpu.SemaphoreType.DMA] * 2,
          [pltpu.SemaphoreType.DMA] * 2,
          pltpu.SemaphoreType.DMA,
      ],
  )
  def k(td_hbm, ts_hbm, dst_hbm, src_hbm, gd_hbm, gs_hbm,
        idxd, idxs, rowsd, rowss, taild, tails,
        gsemd, gsems, wsemd, wsems, tsem):
    wid = lax.axis_index("s") * _NC + lax.axis_index("c")
    base = wid * _EW
    pltpu.sync_copy(dst_hbm.at[pl.ds(base, _EW)], idxd)
    pltpu.sync_copy(src_hbm.at[pl.ds(base, _EW)], idxs)

    def body(g, carry):
      gds = []
      for b in range(2):
        c = 2 * g + b
        osl = pl.ds(base + c * _CH, _CH)
        isl = pl.ds(c * _CH, _CH)

        @pl.when(g > 0)
        def _():
          pltpu.make_async_copy(rowsd[b], gd_hbm.at[osl], wsemd[b]).wait()
          pltpu.make_async_copy(rowss[b], gs_hbm.at[osl], wsems[b]).wait()

        gds.append((
            pltpu.async_copy(td_hbm.at[idxd.at[isl]], rowsd[b], gsemd[b]),
            pltpu.async_copy(ts_hbm.at[idxs.at[isl]], rowss[b], gsems[b]),
        ))
      for b in range(2):
        c = 2 * g + b
        osl = pl.ds(base + c * _CH, _CH)
        cpd, cps = gds[b]
        cpd.wait()
        cps.wait()
        pltpu.async_copy(rowsd[b], gd_hbm.at[osl], wsemd[b])
        pltpu.async_copy(rowss[b], gs_hbm.at[osl], wsems[b])
      return carry

    lax.fori_loop(0, _NPAIR, body, 0)

    osl = pl.ds(base + _CREM * _CH, _CH)
    pltpu.make_async_copy(rowsd[0], gd_hbm.at[osl], wsemd[0]).wait()
    pltpu.make_async_copy(rowss[0], gs_hbm.at[osl], wsems[0]).wait()
    isl = pl.ds(_CREM * _CH, _CH)
    cpd = pltpu.async_copy(td_hbm.at[idxd.at[isl]], rowsd[0], gsemd[0])
    cps = pltpu.async_copy(ts_hbm.at[idxs.at[isl]], rowss[0], gsems[0])
    cpd.wait()
    cps.wait()
    pltpu.sync_copy(rowsd[0], gd_hbm.at[osl])
    pltpu.sync_copy(rowss[0], gs_hbm.at[osl])

    tsl = pl.ds(base + _TOFF, _TAIL)
    cpd = pltpu.async_copy(td_hbm.at[idxd.at[pl.ds(_TOFF, _TAIL)]], taild, tsem)
    cpd.wait()
    cps = pltpu.async_copy(ts_hbm.at[idxs.at[pl.ds(_TOFF, _TAIL)]], tails, tsem)
    cps.wait()
    pltpu.sync_copy(taild, gd_hbm.at[tsl])
    pltpu.sync_copy(tails, gs_hbm.at[tsl])
    osl = pl.ds(base + (_CREM - 1) * _CH, _CH)
    pltpu.make_async_copy(rowsd[1], gd_hbm.at[osl], wsemd[1]).wait()
    pltpu.make_async_copy(rowss[1], gs_hbm.at[osl], wsems[1]).wait()

  return k(td, ts, dst, src)


def _sc_scatter(msg, dst, zeros):
  @functools.partial(
      pl.kernel,
      mesh=_sc_mesh(),
      out_type=jax.ShapeDtypeStruct((_NC, _N, _C), jnp.float32),
      scratch_types=[
          [pltpu.VMEM((_CH,), jnp.int32)] * 2,
          [pltpu.VMEM((_CH, _C), jnp.float32)] * 2,
          pltpu.VMEM((_TAIL,), jnp.int32),
          pltpu.VMEM((_TAIL, _C), jnp.float32),
          pltpu.VMEM_SHARED((_N, _C), jnp.float32),
          [pltpu.SemaphoreType.DMA] * 2,
          [pltpu.SemaphoreType.DMA] * 2,
          [pltpu.SemaphoreType.DMA] * 2,
          pltpu.SemaphoreType.DMA,
      ],
  )
  def k(msg_hbm, dst_hbm, z_hbm, out_hbm, idx, rows, idxt, rowst, acc,
        isem, lsem, ssem, tsem):
    cid = lax.axis_index("c")
    sid = lax.axis_index("s")
    wid = sid * _NC + cid

    @pl.when(sid == 0)
    def _():
      pltpu.sync_copy(z_hbm, acc)

    plsc.subcore_barrier()
    base = wid * _EW

    def body(g, carry):
      lds = []
      for b in range(2):
        c = 2 * g + b
        osl = pl.ds(base + c * _CH, _CH)

        @pl.when(g > 0)
        def _():
          pltpu.make_async_copy(rows[b], acc.at[idx[b]], ssem[b]).wait()

        lds.append((
            pltpu.async_copy(dst_hbm.at[osl], idx[b], isem[b]),
            pltpu.async_copy(msg_hbm.at[osl], rows[b], lsem[b]),
        ))
      for b in range(2):
        cpi, cpm = lds[b]
        cpi.wait()
        cpm.wait()
        pltpu.async_copy(rows[b], acc.at[idx[b]], ssem[b], add=True)
      return carry

    lax.fori_loop(0, _NPAIR, body, 0)

    pltpu.make_async_copy(rows[0], acc.at[idx[0]], ssem[0]).wait()
    osl = pl.ds(base + _CREM * _CH, _CH)
    cpi = pltpu.async_copy(dst_hbm.at[osl], idx[0], isem[0])
    cpm = pltpu.async_copy(msg_hbm.at[osl], rows[0], lsem[0])
    cpi.wait()
    cpm.wait()
    pltpu.sync_copy(rows[0], acc.at[idx[0]], add=True)

    pltpu.make_async_copy(rows[1], acc.at[idx[1]], ssem[1]).wait()
    tsl = pl.ds(base + _TOFF, _TAIL)
    cpi = pltpu.async_copy(dst_hbm.at[tsl], idxt, tsem)
    cpi.wait()
    cpm = pltpu.async_copy(msg_hbm.at[tsl], rowst, tsem)
    cpm.wait()
    pltpu.sync_copy(rowst, acc.at[idxt], add=True)

    plsc.subcore_barrier()

    @pl.when(sid == 0)
    def _():
      pltpu.sync_copy(acc, out_hbm.at[cid])

  return k(msg, dst, zeros)


def _rne_bf16_bits(v):
  bits = lax.bitcast_convert_type(v, jnp.int32)
  return (bits + 0x7FFF + ((bits >> 16) & 1)) >> 16


def _pack2(f, s):
  return (_rne_bf16_bits(s) << 16) | (_rne_bf16_bits(f) & 0xFFFF)


def _unpack_lo(w):
  return lax.bitcast_convert_type(w << 16, jnp.float32)


def _unpack_hi(w):
  return lax.bitcast_convert_type(w & jnp.int32(-65536), jnp.float32)


def _proj(x, wd, ws):
  def body(x_ref, wd_ref, ws_ref, td_ref, ts_ref):
    xv = x_ref[...]
    pd = jnp.dot(xv, wd_ref[...], preferred_element_type=jnp.float32, precision=lax.Precision.HIGHEST)
    ps = jnp.dot(xv, ws_ref[...], preferred_element_type=jnp.float32, precision=lax.Precision.HIGHEST)
    td_ref[...] = _pack2(pd[:, :_C], pd[:, _C:])
    ts_ref[...] = _pack2(ps[:, :_C], ps[:, _C:])

  return pl.pallas_call(
      body,
      grid=(_N // _TN,),
      in_specs=[
          pl.BlockSpec((_TN, _C), lambda i: (i, 0)),
          pl.BlockSpec((_C, 2 * _C), lambda i: (0, 0)),
          pl.BlockSpec((_C, 2 * _C), lambda i: (0, 0)),
      ],
      out_specs=[
          pl.BlockSpec((_TN, _C), lambda i: (i, 0)),
          pl.BlockSpec((_TN, _C), lambda i: (i, 0)),
      ],
      out_shape=[jax.ShapeDtypeStruct((_N, _C), jnp.int32)] * 2,
  )(x, wd, ws)


def _edge(gd, gs, ea, wfe, wse, bf8, bs8):
  def body(gd_ref, gs_ref, ea_ref, wfe_ref, wse_ref, bf_ref, bs_ref, o_ref):
    eav = ea_ref[...]
    ef = jnp.dot(eav, wfe_ref[...], preferred_element_type=jnp.float32, precision=lax.Precision.HIGHEST)
    es = jnp.dot(eav, wse_ref[...], preferred_element_type=jnp.float32, precision=lax.Precision.HIGHEST)
    gd = gd_ref[...]
    gs = gs_ref[...]
    a = _unpack_lo(gd) + _unpack_lo(gs) + ef + bf_ref[0:1, :]
    b = _unpack_hi(gd) + _unpack_hi(gs) + es + bs_ref[0:1, :]
    gate = 1.0 / (1.0 + jnp.exp(-a))
    core = jnp.maximum(b, 0.0) + jnp.log1p(jnp.exp(-jnp.abs(b)))
    o_ref[...] = gate * core

  ne = gd.shape[0]
  return pl.pallas_call(
      body,
      grid=(ne // _TE,),
      in_specs=[
          pl.BlockSpec((_TE, _C), lambda i: (i, 0)),
          pl.BlockSpec((_TE, _C), lambda i: (i, 0)),
          pl.BlockSpec((_TE, _D), lambda i: (i, 0)),
          pl.BlockSpec((_D, _C), lambda i: (0, 0)),
          pl.BlockSpec((_D, _C), lambda i: (0, 0)),
          pl.BlockSpec((8, _C), lambda i: (0, 0)),
          pl.BlockSpec((8, _C), lambda i: (0, 0)),
      ],
      out_specs=pl.BlockSpec((_TE, _C), lambda i: (i, 0)),
      out_shape=jax.ShapeDtypeStruct((ne, _C), jnp.float32),
  )(gd, gs, ea, wfe, wse, bf8, bs8)


def _mlp1(x, pa, pb, w1, b18):
  def body(x_ref, pa_ref, pb_ref, w1_ref, b1_ref, x1_ref, h_ref, s_ref):
    i = pl.program_id(0)
    x1 = (x_ref[...] + (pa_ref[0] + pa_ref[1]) + (pb_ref[0] + pb_ref[1]))
    x1_ref[...] = x1
    h = jnp.dot(x1, w1_ref[...], preferred_element_type=jnp.float32, precision=lax.Precision.HIGHEST)
    h = h + b1_ref[0:1, :]
    h_ref[...] = h
    upd = jnp.concatenate(
        [
            jnp.sum(h, axis=0, keepdims=True),
            jnp.sum(h * h, axis=0, keepdims=True),
            jnp.zeros((6, _H), jnp.float32),
        ],
        axis=0,
    )

    @pl.when(i == 0)
    def _():
      s_ref[...] = upd

    @pl.when(i > 0)
    def _():
      s_ref[...] += upd

  return pl.pallas_call(
      body,
      grid=(_N // _TN,),
      in_specs=[
          pl.BlockSpec((_TN, _C), lambda i: (i, 0)),
          pl.BlockSpec((2, _TN, _C), lambda i: (0, i, 0)),
          pl.BlockSpec((2, _TN, _C), lambda i: (0, i, 0)),
          pl.BlockSpec((_C, _H), lambda i: (0, 0)),
          pl.BlockSpec((8, _H), lambda i: (0, 0)),
      ],
      out_specs=[
          pl.BlockSpec((_TN, _C), lambda i: (i, 0)),
          pl.BlockSpec((_TN, _H), lambda i: (i, 0)),
          pl.BlockSpec((8, _H), lambda i: (0, 0)),
      ],
      out_shape=[
          jax.ShapeDtypeStruct((_N, _C), jnp.float32),
          jax.ShapeDtypeStruct((_N, _H), jnp.float32),
          jax.ShapeDtypeStruct((8, _H), jnp.float32),
      ],
  )(x, pa, pb, w1, b18)


def _mlp2(h, x1, stats, onehot, bnw8, bnb8, w2, b28):
  def body(h_ref, x1_ref, s_ref, oh_ref, bnw_ref, bnb_ref, w2_ref, b2_ref,
           x2_ref, seg_ref):
    i = pl.program_id(0)
    mu = s_ref[0:1, :] / _N
    var = s_ref[1:2, :] / _N - mu * mu
    hn = (h_ref[...] - mu) * lax.rsqrt(var + _EPS)
    hn = hn * bnw_ref[0:1, :] + bnb_ref[0:1, :]
    hr = jnp.maximum(hn, 0.0)
    xp = jnp.dot(hr, w2_ref[...], preferred_element_type=jnp.float32, precision=lax.Precision.HIGHEST)
    x2 = x1_ref[...] + xp + b2_ref[0:1, :]
    x2_ref[...] = x2
    oh = oh_ref[...]
    dn = (((0,), (0,)), ((), ()))
    s1 = lax.dot_general(oh, x2, dn, preferred_element_type=jnp.float32, precision=lax.Precision.HIGHEST)
    s2 = lax.dot_general(oh, x2 * x2, dn, preferred_element_type=jnp.float32, precision=lax.Precision.HIGHEST)
    dg = lax.dot_general(oh, jnp.ones_like(x2), dn,
                         preferred_element_type=jnp.float32, precision=lax.Precision.HIGHEST)
    upd = jnp.concatenate([s1, s2, dg], axis=0)

    @pl.when(i == 0)
    def _():
      seg_ref[...] = upd

    @pl.when(i > 0)
    def _():
      seg_ref[...] += upd

  return pl.pallas_call(
      body,
      grid=(_N // _TN,),
      in_specs=[
          pl.BlockSpec((_TN, _H), lambda i: (i, 0)),
          pl.BlockSpec((_TN, _C), lambda i: (i, 0)),
          pl.BlockSpec((8, _H), lambda i: (0, 0)),
          pl.BlockSpec((_TN, _G), lambda i: (i, 0)),
          pl.BlockSpec((8, _H), lambda i: (0, 0)),
          pl.BlockSpec((8, _H), lambda i: (0, 0)),
          pl.BlockSpec((_H, _C), lambda i: (0, 0)),
          pl.BlockSpec((8, _C), lambda i: (0, 0)),
      ],
      out_specs=[
          pl.BlockSpec((_TN, _C), lambda i: (i, 0)),
          pl.BlockSpec((3 * _G, _C), lambda i: (0, 0)),
      ],
      out_shape=[
          jax.ShapeDtypeStruct((_N, _C), jnp.float32),
          jax.ShapeDtypeStruct((3 * _G, _C), jnp.float32),
      ],
  )(h, x1, stats, onehot, bnw8, bnb8, w2, b28)


def _gln(x2, onehot, seg, lnw8, lnb8):
  def body(x2_ref, oh_ref, seg_ref, lnw_ref, lnb_ref, o_ref):
    s1 = seg_ref[0:_G, :]
    s2 = seg_ref[_G:2 * _G, :]
    deg = seg_ref[2 * _G:3 * _G, 0:1]
    norm = jnp.maximum(deg, 1.0) * _C
    mean_g = jnp.sum(s1, axis=1, keepdims=True) / norm
    var_g = jnp.sum(s2, axis=1, keepdims=True) / norm - mean_g * mean_g
    inv_g = lax.rsqrt(var_g + _EPS)
    mean_b = jnp.broadcast_to(mean_g, (_G, _C))
    inv_b = jnp.broadcast_to(inv_g, (_G, _C))
    oh = oh_ref[...]
    m = jnp.dot(oh, mean_b, preferred_element_type=jnp.float32, precision=lax.Precision.HIGHEST)
    iv = jnp.dot(oh, inv_b, preferred_element_type=jnp.float32, precision=lax.Precision.HIGHEST)
    o_ref[...] = (x2_ref[...] - m) * iv * lnw_ref[0:1, :] + lnb_ref[0:1, :]

  return pl.pallas_call(
      body,
      grid=(_N // _TN,),
      in_specs=[
          pl.BlockSpec((_TN, _C), lambda i: (i, 0)),
          pl.BlockSpec((_TN, _G), lambda i: (i, 0)),
          pl.BlockSpec((3 * _G, _C), lambda i: (0, 0)),
          pl.BlockSpec((8, _C), lambda i: (0, 0)),
          pl.BlockSpec((8, _C), lambda i: (0, 0)),
      ],
      out_specs=pl.BlockSpec((_TN, _C), lambda i: (i, 0)),
      out_shape=jax.ShapeDtypeStruct((_N, _C), jnp.float32),
  )(x2, onehot, seg, lnw8, lnb8)


def _r8(v):
  return jnp.tile(v.reshape(1, -1), (8, 1))


def kernel(x, node_batch, edge_index, edge_attr, Wf, bf, Ws, bs, W1, b1,
           bn_w, bn_b, W2, b2, ln_w, ln_b):
  src = edge_index[0].astype(jnp.int32)
  dst = edge_index[1].astype(jnp.int32)
  nb = node_batch.astype(jnp.int32)
  onehot = (nb[:, None] == jnp.arange(_G, dtype=jnp.int32)[None, :])
  onehot = onehot.astype(jnp.float32)
  zeros = jnp.zeros((_N, _C), jnp.float32)
  dsta, dstb = dst[:_EH], dst[_EH:]
  srca, srcb = src[:_EH], src[_EH:]
  eaa, eab = edge_attr[:_EH], edge_attr[_EH:]

  for l in range(_L):
    wd = jnp.concatenate([Wf[l, :_C], Ws[l, :_C]], axis=1)
    wsr = jnp.concatenate([Wf[l, _C:2 * _C], Ws[l, _C:2 * _C]], axis=1)
    wfe = Wf[l, 2 * _C:]
    wse = Ws[l, 2 * _C:]
    bf8, bs8 = _r8(bf[l]), _r8(bs[l])

    td, ts = _proj(x, wd, wsr)
    gda, gsa = _sc_gather(td, ts, dsta, srca)
    gdb, gsb = _sc_gather(td, ts, dstb, srcb)
    msga = _edge(gda, gsa, eaa, wfe, wse, bf8, bs8)
    msgb = _edge(gdb, gsb, eab, wfe, wse, bf8, bs8)
    pa = _sc_scatter(msga, dsta, zeros)
    pb = _sc_scatter(msgb, dstb, zeros)
    x1, h, stats = _mlp1(x, pa, pb, W1[l], _r8(b1[l]))
    x2, seg = _mlp2(h, x1, stats, onehot, _r8(bn_w[l]), _r8(bn_b[l]),
                    W2[l], _r8(b2[l]))
    x = _gln(x2, onehot, seg, _r8(ln_w[l]), _r8(ln_b[l]))
  return x


# speedup vs baseline: 3.2910x; 1.0022x over previous
import functools

import jax
import jax.numpy as jnp
from jax import lax
from jax.experimental import pallas as pl
from jax.experimental.pallas import tpu as pltpu
from jax.experimental.pallas import tpu_sc as plsc

_L = 3
_C = 128
_D = 16
_H = 4 * _C
_N = 10000
_E = 320000
_G = 16
_EPS = 1e-5

_NC = 2
_NS = 16
_NW = _NC * _NS
_EH = _E // 2
_EW = _EH // _NW
_CH = 128
_NFULL = _EW // _CH
_NPAIR = _NFULL // 2
_CREM = 2 * _NPAIR
_TAIL = _EW - _NFULL * _CH
_TOFF = _NFULL * _CH

_TN = 1000
_TE = 2000


def _sc_mesh():
  return plsc.VectorSubcoreMesh(core_axis_name="c", subcore_axis_name="s")


def _sc_gather(td, ts, dst, src):
  @functools.partial(
      pl.kernel,
      mesh=_sc_mesh(),
      out_type=(
          jax.ShapeDtypeStruct((_EH, _C), jnp.int32),
          jax.ShapeDtypeStruct((_EH, _C), jnp.int32),
      ),
      scratch_types=[
          pltpu.VMEM((_EW,), jnp.int32),
          pltpu.VMEM((_EW,), jnp.int32),
          [pltpu.VMEM((_CH, _C), jnp.int32)] * 2,
          [pltpu.VMEM((_CH, _C), jnp.int32)] * 2,
          pltpu.VMEM((_TAIL, _C), jnp.int32),
          pltpu.VMEM((_TAIL, _C), jnp.int32),
          [pltpu.SemaphoreType.DMA] * 2,
          [pltpu.SemaphoreType.DMA] * 2,
          [pltpu.SemaphoreType.DMA] * 2,
          [pltpu.SemaphoreType.DMA] * 2,
          pltpu.SemaphoreType.DMA,
      ],
  )
  def k(td_hbm, ts_hbm, dst_hbm, src_hbm, gd_hbm, gs_hbm,
        idxd, idxs, rowsd, rowss, taild, tails,
        gsemd, gsems, wsemd, wsems, tsem):
    wid = lax.axis_index("s") * _NC + lax.axis_index("c")
    base = wid * _EW
    pltpu.sync_copy(dst_hbm.at[pl.ds(base, _EW)], idxd)
    pltpu.sync_copy(src_hbm.at[pl.ds(base, _EW)], idxs)

    def body(g, carry):
      gds = []
      for b in range(2):
        c = 2 * g + b
        osl = pl.ds(base + c * _CH, _CH)
        isl = pl.ds(c * _CH, _CH)

        @pl.when(g > 0)
        def _():
          pltpu.make_async_copy(rowsd[b], gd_hbm.at[osl], wsemd[b]).wait()
          pltpu.make_async_copy(rowss[b], gs_hbm.at[osl], wsems[b]).wait()

        gds.append((
            pltpu.async_copy(td_hbm.at[idxd.at[isl]], rowsd[b], gsemd[b]),
            pltpu.async_copy(ts_hbm.at[idxs.at[isl]], rowss[b], gsems[b]),
        ))
      for b in range(2):
        c = 2 * g + b
        osl = pl.ds(base + c * _CH, _CH)
        cpd, cps = gds[b]
        cpd.wait()
        cps.wait()
        pltpu.async_copy(rowsd[b], gd_hbm.at[osl], wsemd[b])
        pltpu.async_copy(rowss[b], gs_hbm.at[osl], wsems[b])
      return carry

    lax.fori_loop(0, _NPAIR, body, 0)

    osl = pl.ds(base + _CREM * _CH, _CH)
    pltpu.make_async_copy(rowsd[0], gd_hbm.at[osl], wsemd[0]).wait()
    pltpu.make_async_copy(rowss[0], gs_hbm.at[osl], wsems[0]).wait()
    isl = pl.ds(_CREM * _CH, _CH)
    cpd = pltpu.async_copy(td_hbm.at[idxd.at[isl]], rowsd[0], gsemd[0])
    cps = pltpu.async_copy(ts_hbm.at[idxs.at[isl]], rowss[0], gsems[0])
    cpd.wait()
    cps.wait()
    pltpu.sync_copy(rowsd[0], gd_hbm.at[osl])
    pltpu.sync_copy(rowss[0], gs_hbm.at[osl])

    tsl = pl.ds(base + _TOFF, _TAIL)
    cpd = pltpu.async_copy(td_hbm.at[idxd.at[pl.ds(_TOFF, _TAIL)]], taild, tsem)
    cpd.wait()
    cps = pltpu.async_copy(ts_hbm.at[idxs.at[pl.ds(_TOFF, _TAIL)]], tails, tsem)
    cps.wait()
    pltpu.sync_copy(taild, gd_hbm.at[tsl])
    pltpu.sync_copy(tails, gs_hbm.at[tsl])
    osl = pl.ds(base + (_CREM - 1) * _CH, _CH)
    pltpu.make_async_copy(rowsd[1], gd_hbm.at[osl], wsemd[1]).wait()
    pltpu.make_async_copy(rowss[1], gs_hbm.at[osl], wsems[1]).wait()

  return k(td, ts, dst, src)


def _sc_scatter(msg, dst, zeros):
  @functools.partial(
      pl.kernel,
      mesh=_sc_mesh(),
      out_type=jax.ShapeDtypeStruct((_NC, _N, _C), jnp.float32),
      scratch_types=[
          [pltpu.VMEM((_CH,), jnp.int32)] * 2,
          [pltpu.VMEM((_CH, _C), jnp.float32)] * 2,
          pltpu.VMEM((_TAIL,), jnp.int32),
          pltpu.VMEM((_TAIL, _C), jnp.float32),
          pltpu.VMEM_SHARED((_N, _C), jnp.float32),
          [pltpu.SemaphoreType.DMA] * 2,
          [pltpu.SemaphoreType.DMA] * 2,
          [pltpu.SemaphoreType.DMA] * 2,
          pltpu.SemaphoreType.DMA,
      ],
  )
  def k(msg_hbm, dst_hbm, z_hbm, out_hbm, idx, rows, idxt, rowst, acc,
        isem, lsem, ssem, tsem):
    cid = lax.axis_index("c")
    sid = lax.axis_index("s")
    wid = sid * _NC + cid

    @pl.when(sid == 0)
    def _():
      pltpu.sync_copy(z_hbm, acc)

    plsc.subcore_barrier()
    base = wid * _EW

    def body(g, carry):
      lds = []
      for b in range(2):
        c = 2 * g + b
        osl = pl.ds(base + c * _CH, _CH)

        @pl.when(g > 0)
        def _():
          pltpu.make_async_copy(rows[b], acc.at[idx[b]], ssem[b]).wait()

        lds.append((
            pltpu.async_copy(dst_hbm.at[osl], idx[b], isem[b]),
            pltpu.async_copy(msg_hbm.at[osl], rows[b], lsem[b]),
        ))
      for b in range(2):
        cpi, cpm = lds[b]
        cpi.wait()
        cpm.wait()
        pltpu.async_copy(rows[b], acc.at[idx[b]], ssem[b], add=True)
      return carry

    lax.fori_loop(0, _NPAIR, body, 0)

    pltpu.make_async_copy(rows[0], acc.at[idx[0]], ssem[0]).wait()
    osl = pl.ds(base + _CREM * _CH, _CH)
    cpi = pltpu.async_copy(dst_hbm.at[osl], idx[0], isem[0])
    cpm = pltpu.async_copy(msg_hbm.at[osl], rows[0], lsem[0])
    cpi.wait()
    cpm.wait()
    pltpu.sync_copy(rows[0], acc.at[idx[0]], add=True)

    pltpu.make_async_copy(rows[1], acc.at[idx[1]], ssem[1]).wait()
    tsl = pl.ds(base + _TOFF, _TAIL)
    cpi = pltpu.async_copy(dst_hbm.at[tsl], idxt, tsem)
    cpi.wait()
    cpm = pltpu.async_copy(msg_hbm.at[tsl], rowst, tsem)
    cpm.wait()
    pltpu.sync_copy(rowst, acc.at[idxt], add=True)

    plsc.subcore_barrier()

    @pl.when(sid == 0)
    def _():
      pltpu.sync_copy(acc, out_hbm.at[cid])

  return k(msg, dst, zeros)


def _rne_bf16_bits(v):
  bits = lax.bitcast_convert_type(v, jnp.int32)
  return (bits + 0x7FFF + ((bits >> 16) & 1)) >> 16


def _pack2(f, s):
  return (_rne_bf16_bits(s) << 16) | (_rne_bf16_bits(f) & 0xFFFF)


def _unpack_lo(w):
  return lax.bitcast_convert_type(w << 16, jnp.float32)


def _unpack_hi(w):
  return lax.bitcast_convert_type(w & jnp.int32(-65536), jnp.float32)


def _proj(x, wd, ws):
  def body(x_ref, wd_ref, ws_ref, td_ref, ts_ref):
    xv = x_ref[...]
    pd = jnp.dot(xv, wd_ref[...], preferred_element_type=jnp.float32, precision=lax.Precision.HIGHEST)
    ps = jnp.dot(xv, ws_ref[...], preferred_element_type=jnp.float32, precision=lax.Precision.HIGHEST)
    td_ref[...] = _pack2(pd[:, :_C], pd[:, _C:])
    ts_ref[...] = _pack2(ps[:, :_C], ps[:, _C:])

  return pl.pallas_call(
      body,
      grid=(_N // _TN,),
      in_specs=[
          pl.BlockSpec((_TN, _C), lambda i: (i, 0)),
          pl.BlockSpec((_C, 2 * _C), lambda i: (0, 0)),
          pl.BlockSpec((_C, 2 * _C), lambda i: (0, 0)),
      ],
      out_specs=[
          pl.BlockSpec((_TN, _C), lambda i: (i, 0)),
          pl.BlockSpec((_TN, _C), lambda i: (i, 0)),
      ],
      out_shape=[jax.ShapeDtypeStruct((_N, _C), jnp.int32)] * 2,
  )(x, wd, ws)


def _edge(gd, gs, ea, wfe, wse, bf8, bs8):
  def body(gd_ref, gs_ref, ea_ref, wfe_ref, wse_ref, bf_ref, bs_ref, o_ref):
    eav = ea_ref[...]
    ef = jnp.dot(eav, wfe_ref[...], preferred_element_type=jnp.float32, precision=lax.Precision.HIGHEST)
    es = jnp.dot(eav, wse_ref[...], preferred_element_type=jnp.float32, precision=lax.Precision.HIGHEST)
    gd = gd_ref[...]
    gs = gs_ref[...]
    a = _unpack_lo(gd) + _unpack_lo(gs) + ef + bf_ref[0:1, :]
    b = _unpack_hi(gd) + _unpack_hi(gs) + es + bs_ref[0:1, :]
    gate = 0.5 + 0.5 * jnp.tanh(0.5 * a)
    core = jnp.maximum(b, 0.0) + jnp.log1p(jnp.exp(-jnp.abs(b)))
    o_ref[...] = gate * core

  ne = gd.shape[0]
  return pl.pallas_call(
      body,
      grid=(ne // _TE,),
      in_specs=[
          pl.BlockSpec((_TE, _C), lambda i: (i, 0)),
          pl.BlockSpec((_TE, _C), lambda i: (i, 0)),
          pl.BlockSpec((_TE, _D), lambda i: (i, 0)),
          pl.BlockSpec((_D, _C), lambda i: (0, 0)),
          pl.BlockSpec((_D, _C), lambda i: (0, 0)),
          pl.BlockSpec((8, _C), lambda i: (0, 0)),
          pl.BlockSpec((8, _C), lambda i: (0, 0)),
      ],
      out_specs=pl.BlockSpec((_TE, _C), lambda i: (i, 0)),
      out_shape=jax.ShapeDtypeStruct((ne, _C), jnp.float32),
  )(gd, gs, ea, wfe, wse, bf8, bs8)


def _mlp1(x, pa, pb, w1, b18):
  def body(x_ref, pa_ref, pb_ref, w1_ref, b1_ref, x1_ref, h_ref, s_ref):
    i = pl.program_id(0)
    x1 = (x_ref[...] + (pa_ref[0] + pa_ref[1]) + (pb_ref[0] + pb_ref[1]))
    x1_ref[...] = x1
    h = jnp.dot(x1, w1_ref[...], preferred_element_type=jnp.float32, precision=lax.Precision.HIGHEST)
    h = h + b1_ref[0:1, :]
    h_ref[...] = h
    upd = jnp.concatenate(
        [
            jnp.sum(h, axis=0, keepdims=True),
            jnp.sum(h * h, axis=0, keepdims=True),
            jnp.zeros((6, _H), jnp.float32),
        ],
        axis=0,
    )

    @pl.when(i == 0)
    def _():
      s_ref[...] = upd

    @pl.when(i > 0)
    def _():
      s_ref[...] += upd

  return pl.pallas_call(
      body,
      grid=(_N // _TN,),
      in_specs=[
          pl.BlockSpec((_TN, _C), lambda i: (i, 0)),
          pl.BlockSpec((2, _TN, _C), lambda i: (0, i, 0)),
          pl.BlockSpec((2, _TN, _C), lambda i: (0, i, 0)),
          pl.BlockSpec((_C, _H), lambda i: (0, 0)),
          pl.BlockSpec((8, _H), lambda i: (0, 0)),
      ],
      out_specs=[
          pl.BlockSpec((_TN, _C), lambda i: (i, 0)),
          pl.BlockSpec((_TN, _H), lambda i: (i, 0)),
          pl.BlockSpec((8, _H), lambda i: (0, 0)),
      ],
      out_shape=[
          jax.ShapeDtypeStruct((_N, _C), jnp.float32),
          jax.ShapeDtypeStruct((_N, _H), jnp.float32),
          jax.ShapeDtypeStruct((8, _H), jnp.float32),
      ],
  )(x, pa, pb, w1, b18)


def _mlp2(h, x1, stats, onehot, bnw8, bnb8, w2, b28):
  def body(h_ref, x1_ref, s_ref, oh_ref, bnw_ref, bnb_ref, w2_ref, b2_ref,
           x2_ref, seg_ref):
    i = pl.program_id(0)
    mu = s_ref[0:1, :] / _N
    var = s_ref[1:2, :] / _N - mu * mu
    hn = (h_ref[...] - mu) * lax.rsqrt(var + _EPS)
    hn = hn * bnw_ref[0:1, :] + bnb_ref[0:1, :]
    hr = jnp.maximum(hn, 0.0)
    xp = jnp.dot(hr, w2_ref[...], preferred_element_type=jnp.float32, precision=lax.Precision.HIGHEST)
    x2 = x1_ref[...] + xp + b2_ref[0:1, :]
    x2_ref[...] = x2
    oh = oh_ref[...]
    dn = (((0,), (0,)), ((), ()))
    s1 = lax.dot_general(oh, x2, dn, preferred_element_type=jnp.float32, precision=lax.Precision.HIGHEST)
    s2 = lax.dot_general(oh, x2 * x2, dn, preferred_element_type=jnp.float32, precision=lax.Precision.HIGHEST)
    dg = lax.dot_general(oh, jnp.ones_like(x2), dn,
                         preferred_element_type=jnp.float32, precision=lax.Precision.HIGHEST)
    upd = jnp.concatenate([s1, s2, dg], axis=0)

    @pl.when(i == 0)
    def _():
      seg_ref[...] = upd

    @pl.when(i > 0)
    def _():
      seg_ref[...] += upd

  return pl.pallas_call(
      body,
      grid=(_N // _TN,),
      in_specs=[
          pl.BlockSpec((_TN, _H), lambda i: (i, 0)),
          pl.BlockSpec((_TN, _C), lambda i: (i, 0)),
          pl.BlockSpec((8, _H), lambda i: (0, 0)),
          pl.BlockSpec((_TN, _G), lambda i: (i, 0)),
          pl.BlockSpec((8, _H), lambda i: (0, 0)),
          pl.BlockSpec((8, _H), lambda i: (0, 0)),
          pl.BlockSpec((_H, _C), lambda i: (0, 0)),
          pl.BlockSpec((8, _C), lambda i: (0, 0)),
      ],
      out_specs=[
          pl.BlockSpec((_TN, _C), lambda i: (i, 0)),
          pl.BlockSpec((3 * _G, _C), lambda i: (0, 0)),
      ],
      out_shape=[
          jax.ShapeDtypeStruct((_N, _C), jnp.float32),
          jax.ShapeDtypeStruct((3 * _G, _C), jnp.float32),
      ],
  )(h, x1, stats, onehot, bnw8, bnb8, w2, b28)


def _gln(x2, onehot, seg, lnw8, lnb8):
  def body(x2_ref, oh_ref, seg_ref, lnw_ref, lnb_ref, o_ref):
    s1 = seg_ref[0:_G, :]
    s2 = seg_ref[_G:2 * _G, :]
    deg = seg_ref[2 * _G:3 * _G, 0:1]
    norm = jnp.maximum(deg, 1.0) * _C
    mean_g = jnp.sum(s1, axis=1, keepdims=True) / norm
    var_g = jnp.sum(s2, axis=1, keepdims=True) / norm - mean_g * mean_g
    inv_g = lax.rsqrt(var_g + _EPS)
    mean_b = jnp.broadcast_to(mean_g, (_G, _C))
    inv_b = jnp.broadcast_to(inv_g, (_G, _C))
    oh = oh_ref[...]
    m = jnp.dot(oh, mean_b, preferred_element_type=jnp.float32, precision=lax.Precision.HIGHEST)
    iv = jnp.dot(oh, inv_b, preferred_element_type=jnp.float32, precision=lax.Precision.HIGHEST)
    o_ref[...] = (x2_ref[...] - m) * iv * lnw_ref[0:1, :] + lnb_ref[0:1, :]

  return pl.pallas_call(
      body,
      grid=(_N // _TN,),
      in_specs=[
          pl.BlockSpec((_TN, _C), lambda i: (i, 0)),
          pl.BlockSpec((_TN, _G), lambda i: (i, 0)),
          pl.BlockSpec((3 * _G, _C), lambda i: (0, 0)),
          pl.BlockSpec((8, _C), lambda i: (0, 0)),
          pl.BlockSpec((8, _C), lambda i: (0, 0)),
      ],
      out_specs=pl.BlockSpec((_TN, _C), lambda i: (i, 0)),
      out_shape=jax.ShapeDtypeStruct((_N, _C), jnp.float32),
  )(x2, onehot, seg, lnw8, lnb8)


def _r8(v):
  return jnp.tile(v.reshape(1, -1), (8, 1))


def kernel(x, node_batch, edge_index, edge_attr, Wf, bf, Ws, bs, W1, b1,
           bn_w, bn_b, W2, b2, ln_w, ln_b):
  src = edge_index[0].astype(jnp.int32)
  dst = edge_index[1].astype(jnp.int32)
  nb = node_batch.astype(jnp.int32)
  onehot = (nb[:, None] == jnp.arange(_G, dtype=jnp.int32)[None, :])
  onehot = onehot.astype(jnp.float32)
  zeros = jnp.zeros((_N, _C), jnp.float32)
  dsta, dstb = dst[:_EH], dst[_EH:]
  srca, srcb = src[:_EH], src[_EH:]
  eaa, eab = edge_attr[:_EH], edge_attr[_EH:]

  for l in range(_L):
    wd = jnp.concatenate([Wf[l, :_C], Ws[l, :_C]], axis=1)
    wsr = jnp.concatenate([Wf[l, _C:2 * _C], Ws[l, _C:2 * _C]], axis=1)
    wfe = Wf[l, 2 * _C:]
    wse = Ws[l, 2 * _C:]
    bf8, bs8 = _r8(bf[l]), _r8(bs[l])

    td, ts = _proj(x, wd, wsr)
    gda, gsa = _sc_gather(td, ts, dsta, srca)
    gdb, gsb = _sc_gather(td, ts, dstb, srcb)
    msga = _edge(gda, gsa, eaa, wfe, wse, bf8, bs8)
    msgb = _edge(gdb, gsb, eab, wfe, wse, bf8, bs8)
    pa = _sc_scatter(msga, dsta, zeros)
    pb = _sc_scatter(msgb, dstb, zeros)
    x1, h, stats = _mlp1(x, pa, pb, W1[l], _r8(b1[l]))
    x2, seg = _mlp2(h, x1, stats, onehot, _r8(bn_w[l]), _r8(bn_b[l]),
                    W2[l], _r8(b2[l]))
    x = _gln(x2, onehot, seg, _r8(ln_w[l]), _r8(ln_b[l]))
  return x


# speedup vs baseline: 3.3166x; 1.0078x over previous
import functools

import jax
import jax.numpy as jnp
from jax import lax
from jax.experimental import pallas as pl
from jax.experimental.pallas import tpu as pltpu
from jax.experimental.pallas import tpu_sc as plsc

_L = 3
_C = 128
_D = 16
_H = 4 * _C
_N = 10000
_E = 320000
_G = 16
_EPS = 1e-5

_NC = 2
_NS = 16
_NW = _NC * _NS
_EH = _E // 2
_EW = _EH // _NW
_CH = 128
_NFULL = _EW // _CH
_NPAIR = _NFULL // 2
_CREM = 2 * _NPAIR
_TAIL = _EW - _NFULL * _CH
_TOFF = _NFULL * _CH

_TN = 1000
_TE = 2000


def _sc_mesh():
  return plsc.VectorSubcoreMesh(core_axis_name="c", subcore_axis_name="s")


def _sc_gather(td, ts, dst, src):
  @functools.partial(
      pl.kernel,
      mesh=_sc_mesh(),
      out_type=(
          jax.ShapeDtypeStruct((_EH, _C), jnp.int32),
          jax.ShapeDtypeStruct((_EH, _C), jnp.int32),
      ),
      scratch_types=[
          pltpu.VMEM((_EW,), jnp.int32),
          pltpu.VMEM((_EW,), jnp.int32),
          [pltpu.VMEM((_CH, _C), jnp.int32)] * 2,
          [pltpu.VMEM((_CH, _C), jnp.int32)] * 2,
          pltpu.VMEM((_TAIL, _C), jnp.int32),
          pltpu.VMEM((_TAIL, _C), jnp.int32),
          [pltpu.SemaphoreType.DMA] * 2,
          [pltpu.SemaphoreType.DMA] * 2,
          [pltpu.SemaphoreType.DMA] * 2,
          [pltpu.SemaphoreType.DMA] * 2,
          pltpu.SemaphoreType.DMA,
      ],
  )
  def k(td_hbm, ts_hbm, dst_hbm, src_hbm, gd_hbm, gs_hbm,
        idxd, idxs, rowsd, rowss, taild, tails,
        gsemd, gsems, wsemd, wsems, tsem):
    wid = lax.axis_index("s") * _NC + lax.axis_index("c")
    base = wid * _EW
    pltpu.sync_copy(dst_hbm.at[pl.ds(base, _EW)], idxd)
    pltpu.sync_copy(src_hbm.at[pl.ds(base, _EW)], idxs)

    def body(g, carry):
      gds = []
      for b in range(2):
        c = 2 * g + b
        osl = pl.ds(base + c * _CH, _CH)
        isl = pl.ds(c * _CH, _CH)

        @pl.when(g > 0)
        def _():
          pltpu.make_async_copy(rowsd[b], gd_hbm.at[osl], wsemd[b]).wait()
          pltpu.make_async_copy(rowss[b], gs_hbm.at[osl], wsems[b]).wait()

        gds.append((
            pltpu.async_copy(td_hbm.at[idxd.at[isl]], rowsd[b], gsemd[b]),
            pltpu.async_copy(ts_hbm.at[idxs.at[isl]], rowss[b], gsems[b]),
        ))
      for b in range(2):
        c = 2 * g + b
        osl = pl.ds(base + c * _CH, _CH)
        cpd, cps = gds[b]
        cpd.wait()
        cps.wait()
        pltpu.async_copy(rowsd[b], gd_hbm.at[osl], wsemd[b])
        pltpu.async_copy(rowss[b], gs_hbm.at[osl], wsems[b])
      return carry

    lax.fori_loop(0, _NPAIR, body, 0)

    osl = pl.ds(base + _CREM * _CH, _CH)
    pltpu.make_async_copy(rowsd[0], gd_hbm.at[osl], wsemd[0]).wait()
    pltpu.make_async_copy(rowss[0], gs_hbm.at[osl], wsems[0]).wait()
    isl = pl.ds(_CREM * _CH, _CH)
    cpd = pltpu.async_copy(td_hbm.at[idxd.at[isl]], rowsd[0], gsemd[0])
    cps = pltpu.async_copy(ts_hbm.at[idxs.at[isl]], rowss[0], gsems[0])
    cpd.wait()
    cps.wait()
    pltpu.sync_copy(rowsd[0], gd_hbm.at[osl])
    pltpu.sync_copy(rowss[0], gs_hbm.at[osl])

    tsl = pl.ds(base + _TOFF, _TAIL)
    cpd = pltpu.async_copy(td_hbm.at[idxd.at[pl.ds(_TOFF, _TAIL)]], taild, tsem)
    cpd.wait()
    cps = pltpu.async_copy(ts_hbm.at[idxs.at[pl.ds(_TOFF, _TAIL)]], tails, tsem)
    cps.wait()
    pltpu.sync_copy(taild, gd_hbm.at[tsl])
    pltpu.sync_copy(tails, gs_hbm.at[tsl])
    osl = pl.ds(base + (_CREM - 1) * _CH, _CH)
    pltpu.make_async_copy(rowsd[1], gd_hbm.at[osl], wsemd[1]).wait()
    pltpu.make_async_copy(rowss[1], gs_hbm.at[osl], wsems[1]).wait()

  return k(td, ts, dst, src)


def _sc_scatter(msg, dst, zeros):
  @functools.partial(
      pl.kernel,
      mesh=_sc_mesh(),
      out_type=jax.ShapeDtypeStruct((_NC, _N, _C), jnp.float32),
      scratch_types=[
          [pltpu.VMEM((_CH,), jnp.int32)] * 2,
          [pltpu.VMEM((_CH, _C), jnp.float32)] * 2,
          pltpu.VMEM((_TAIL,), jnp.int32),
          pltpu.VMEM((_TAIL, _C), jnp.float32),
          pltpu.VMEM_SHARED((_N, _C), jnp.float32),
          [pltpu.SemaphoreType.DMA] * 2,
          [pltpu.SemaphoreType.DMA] * 2,
          [pltpu.SemaphoreType.DMA] * 2,
          pltpu.SemaphoreType.DMA,
      ],
  )
  def k(msg_hbm, dst_hbm, z_hbm, out_hbm, idx, rows, idxt, rowst, acc,
        isem, lsem, ssem, tsem):
    cid = lax.axis_index("c")
    sid = lax.axis_index("s")
    wid = sid * _NC + cid

    @pl.when(sid == 0)
    def _():
      pltpu.sync_copy(z_hbm, acc)

    plsc.subcore_barrier()
    base = wid * _EW

    def body(g, carry):
      lds = []
      for b in range(2):
        c = 2 * g + b
        osl = pl.ds(base + c * _CH, _CH)

        @pl.when(g > 0)
        def _():
          pltpu.make_async_copy(rows[b], acc.at[idx[b]], ssem[b]).wait()

        lds.append((
            pltpu.async_copy(dst_hbm.at[osl], idx[b], isem[b]),
            pltpu.async_copy(msg_hbm.at[osl], rows[b], lsem[b]),
        ))
      for b in range(2):
        cpi, cpm = lds[b]
        cpi.wait()
        cpm.wait()
        pltpu.async_copy(rows[b], acc.at[idx[b]], ssem[b], add=True)
      return carry

    lax.fori_loop(0, _NPAIR, body, 0)

    pltpu.make_async_copy(rows[0], acc.at[idx[0]], ssem[0]).wait()
    osl = pl.ds(base + _CREM * _CH, _CH)
    cpi = pltpu.async_copy(dst_hbm.at[osl], idx[0], isem[0])
    cpm = pltpu.async_copy(msg_hbm.at[osl], rows[0], lsem[0])
    cpi.wait()
    cpm.wait()
    pltpu.sync_copy(rows[0], acc.at[idx[0]], add=True)

    pltpu.make_async_copy(rows[1], acc.at[idx[1]], ssem[1]).wait()
    tsl = pl.ds(base + _TOFF, _TAIL)
    cpi = pltpu.async_copy(dst_hbm.at[tsl], idxt, tsem)
    cpi.wait()
    cpm = pltpu.async_copy(msg_hbm.at[tsl], rowst, tsem)
    cpm.wait()
    pltpu.sync_copy(rowst, acc.at[idxt], add=True)

    plsc.subcore_barrier()

    @pl.when(sid == 0)
    def _():
      pltpu.sync_copy(acc, out_hbm.at[cid])

  return k(msg, dst, zeros)


def _rne_bf16_bits(v):
  bits = lax.bitcast_convert_type(v, jnp.int32)
  return (bits + 0x7FFF + ((bits >> 16) & 1)) >> 16


def _pack2(f, s):
  return (_rne_bf16_bits(s) << 16) | (_rne_bf16_bits(f) & 0xFFFF)


def _unpack_lo(w):
  return lax.bitcast_convert_type(w << 16, jnp.float32)


def _unpack_hi(w):
  return lax.bitcast_convert_type(w & jnp.int32(-65536), jnp.float32)


def _proj(x, wd, ws):
  def body(x_ref, wd_ref, ws_ref, td_ref, ts_ref):
    xv = x_ref[...]
    pd = jnp.dot(xv, wd_ref[...], preferred_element_type=jnp.float32, precision=lax.Precision.HIGHEST)
    ps = jnp.dot(xv, ws_ref[...], preferred_element_type=jnp.float32, precision=lax.Precision.HIGHEST)
    td_ref[...] = _pack2(pd[:, :_C], pd[:, _C:])
    ts_ref[...] = _pack2(ps[:, :_C], ps[:, _C:])

  return pl.pallas_call(
      body,
      grid=(_N // _TN,),
      in_specs=[
          pl.BlockSpec((_TN, _C), lambda i: (i, 0)),
          pl.BlockSpec((_C, 2 * _C), lambda i: (0, 0)),
          pl.BlockSpec((_C, 2 * _C), lambda i: (0, 0)),
      ],
      out_specs=[
          pl.BlockSpec((_TN, _C), lambda i: (i, 0)),
          pl.BlockSpec((_TN, _C), lambda i: (i, 0)),
      ],
      out_shape=[jax.ShapeDtypeStruct((_N, _C), jnp.int32)] * 2,
  )(x, wd, ws)


def _edge(gd, gs, ea, wfe, wse, bf8, bs8):
  def body(gd_ref, gs_ref, ea_ref, wfe_ref, wse_ref, bf_ref, bs_ref, o_ref):
    eav = ea_ref[...]
    ef = jnp.dot(eav, wfe_ref[...], preferred_element_type=jnp.float32, precision=lax.Precision.HIGHEST)
    es = jnp.dot(eav, wse_ref[...], preferred_element_type=jnp.float32, precision=lax.Precision.HIGHEST)
    gd = gd_ref[...]
    gs = gs_ref[...]
    a = _unpack_lo(gd) + _unpack_lo(gs) + ef + bf_ref[0:1, :]
    b = _unpack_hi(gd) + _unpack_hi(gs) + es + bs_ref[0:1, :]
    gate = 0.5 + 0.5 * jnp.tanh(0.5 * a)
    core = jnp.maximum(b, 0.0) + jnp.log1p(jnp.exp(-jnp.abs(b)))
    o_ref[...] = gate * core

  ne = gd.shape[0]
  return pl.pallas_call(
      body,
      grid=(ne // _TE,),
      in_specs=[
          pl.BlockSpec((_TE, _C), lambda i: (i, 0)),
          pl.BlockSpec((_TE, _C), lambda i: (i, 0)),
          pl.BlockSpec((_TE, _D), lambda i: (i, 0)),
          pl.BlockSpec((_D, _C), lambda i: (0, 0)),
          pl.BlockSpec((_D, _C), lambda i: (0, 0)),
          pl.BlockSpec((8, _C), lambda i: (0, 0)),
          pl.BlockSpec((8, _C), lambda i: (0, 0)),
      ],
      out_specs=pl.BlockSpec((_TE, _C), lambda i: (i, 0)),
      out_shape=jax.ShapeDtypeStruct((ne, _C), jnp.float32),
  )(gd, gs, ea, wfe, wse, bf8, bs8)


def _mlp1(x, pa, pb, w1, b18):
  def body(x_ref, pa_ref, pb_ref, w1_ref, b1_ref, x1_ref, h_ref, s_ref):
    i = pl.program_id(0)
    x1 = (x_ref[...] + (pa_ref[0] + pa_ref[1]) + (pb_ref[0] + pb_ref[1]))
    x1_ref[...] = x1
    h = jnp.dot(x1, w1_ref[...], preferred_element_type=jnp.float32, precision=lax.Precision.HIGHEST)
    h = h + b1_ref[0:1, :]
    h_ref[...] = h
    upd = jnp.concatenate(
        [
            jnp.sum(h, axis=0, keepdims=True),
            jnp.sum(h * h, axis=0, keepdims=True),
            jnp.zeros((6, _H), jnp.float32),
        ],
        axis=0,
    )

    @pl.when(i == 0)
    def _():
      s_ref[...] = upd

    @pl.when(i > 0)
    def _():
      s_ref[...] += upd

  return pl.pallas_call(
      body,
      grid=(_N // _TN,),
      in_specs=[
          pl.BlockSpec((_TN, _C), lambda i: (i, 0)),
          pl.BlockSpec((2, _TN, _C), lambda i: (0, i, 0)),
          pl.BlockSpec((2, _TN, _C), lambda i: (0, i, 0)),
          pl.BlockSpec((_C, _H), lambda i: (0, 0)),
          pl.BlockSpec((8, _H), lambda i: (0, 0)),
      ],
      out_specs=[
          pl.BlockSpec((_TN, _C), lambda i: (i, 0)),
          pl.BlockSpec((_TN, _H), lambda i: (i, 0)),
          pl.BlockSpec((8, _H), lambda i: (0, 0)),
      ],
      out_shape=[
          jax.ShapeDtypeStruct((_N, _C), jnp.float32),
          jax.ShapeDtypeStruct((_N, _H), jnp.float32),
          jax.ShapeDtypeStruct((8, _H), jnp.float32),
      ],
  )(x, pa, pb, w1, b18)


def _mlp2(h, x1, stats, onehot, bnw8, bnb8, w2, b28):
  def body(h_ref, x1_ref, s_ref, oh_ref, bnw_ref, bnb_ref, w2_ref, b2_ref,
           x2_ref, seg_ref):
    i = pl.program_id(0)
    mu = s_ref[0:1, :] / _N
    var = s_ref[1:2, :] / _N - mu * mu
    hn = (h_ref[...] - mu) * lax.rsqrt(var + _EPS)
    hn = hn * bnw_ref[0:1, :] + bnb_ref[0:1, :]
    hr = jnp.maximum(hn, 0.0)
    xp = jnp.dot(hr, w2_ref[...], preferred_element_type=jnp.float32, precision=lax.Precision.HIGHEST)
    x2 = x1_ref[...] + xp + b2_ref[0:1, :]
    x2_ref[...] = x2
    oh = oh_ref[...]
    dn = (((0,), (0,)), ((), ()))
    s1 = lax.dot_general(oh, x2, dn, preferred_element_type=jnp.float32, precision=lax.Precision.HIGHEST)
    s2 = lax.dot_general(oh, x2 * x2, dn, preferred_element_type=jnp.float32, precision=lax.Precision.HIGHEST)
    dg = lax.dot_general(oh, jnp.ones_like(x2), dn,
                         preferred_element_type=jnp.float32, precision=lax.Precision.HIGHEST)
    upd = jnp.concatenate([s1, s2, dg], axis=0)

    @pl.when(i == 0)
    def _():
      seg_ref[...] = upd

    @pl.when(i > 0)
    def _():
      seg_ref[...] += upd

  return pl.pallas_call(
      body,
      grid=(_N // _TN,),
      in_specs=[
          pl.BlockSpec((_TN, _H), lambda i: (i, 0)),
          pl.BlockSpec((_TN, _C), lambda i: (i, 0)),
          pl.BlockSpec((8, _H), lambda i: (0, 0)),
          pl.BlockSpec((_TN, _G), lambda i: (i, 0)),
          pl.BlockSpec((8, _H), lambda i: (0, 0)),
          pl.BlockSpec((8, _H), lambda i: (0, 0)),
          pl.BlockSpec((_H, _C), lambda i: (0, 0)),
          pl.BlockSpec((8, _C), lambda i: (0, 0)),
      ],
      out_specs=[
          pl.BlockSpec((_TN, _C), lambda i: (i, 0)),
          pl.BlockSpec((3 * _G, _C), lambda i: (0, 0)),
      ],
      out_shape=[
          jax.ShapeDtypeStruct((_N, _C), jnp.float32),
          jax.ShapeDtypeStruct((3 * _G, _C), jnp.float32),
      ],
  )(h, x1, stats, onehot, bnw8, bnb8, w2, b28)


def _gln(x2, onehot, seg, lnw8, lnb8, wd=None, ws=None):
  fuse = wd is not None

  def body(x2_ref, oh_ref, seg_ref, lnw_ref, lnb_ref, *rest):
    s1 = seg_ref[0:_G, :]
    s2 = seg_ref[_G:2 * _G, :]
    deg = seg_ref[2 * _G:3 * _G, 0:1]
    norm = jnp.maximum(deg, 1.0) * _C
    mean_g = jnp.sum(s1, axis=1, keepdims=True) / norm
    var_g = jnp.sum(s2, axis=1, keepdims=True) / norm - mean_g * mean_g
    inv_g = lax.rsqrt(var_g + _EPS)
    mean_b = jnp.broadcast_to(mean_g, (_G, _C))
    inv_b = jnp.broadcast_to(inv_g, (_G, _C))
    oh = oh_ref[...]
    m = jnp.dot(oh, mean_b, preferred_element_type=jnp.float32, precision=lax.Precision.HIGHEST)
    iv = jnp.dot(oh, inv_b, preferred_element_type=jnp.float32, precision=lax.Precision.HIGHEST)
    xo = (x2_ref[...] - m) * iv * lnw_ref[0:1, :] + lnb_ref[0:1, :]
    if fuse:
      wd_ref, ws_ref, o_ref, td_ref, ts_ref = rest
      o_ref[...] = xo
      pd = jnp.dot(xo, wd_ref[...], preferred_element_type=jnp.float32, precision=lax.Precision.HIGHEST)
      ps = jnp.dot(xo, ws_ref[...], preferred_element_type=jnp.float32, precision=lax.Precision.HIGHEST)
      td_ref[...] = _pack2(pd[:, :_C], pd[:, _C:])
      ts_ref[...] = _pack2(ps[:, :_C], ps[:, _C:])
    else:
      rest[0][...] = xo

  in_specs = [
      pl.BlockSpec((_TN, _C), lambda i: (i, 0)),
      pl.BlockSpec((_TN, _G), lambda i: (i, 0)),
      pl.BlockSpec((3 * _G, _C), lambda i: (0, 0)),
      pl.BlockSpec((8, _C), lambda i: (0, 0)),
      pl.BlockSpec((8, _C), lambda i: (0, 0)),
  ]
  args = [x2, onehot, seg, lnw8, lnb8]
  out_specs = [pl.BlockSpec((_TN, _C), lambda i: (i, 0))]
  out_shape = [jax.ShapeDtypeStruct((_N, _C), jnp.float32)]
  if fuse:
    in_specs += [pl.BlockSpec((_C, 2 * _C), lambda i: (0, 0))] * 2
    args += [wd, ws]
    out_specs += [pl.BlockSpec((_TN, _C), lambda i: (i, 0))] * 2
    out_shape += [jax.ShapeDtypeStruct((_N, _C), jnp.int32)] * 2
  res = pl.pallas_call(
      body,
      grid=(_N // _TN,),
      in_specs=in_specs,
      out_specs=out_specs,
      out_shape=out_shape,
  )(*args)
  return res if fuse else res[0]


def _r8(v):
  return jnp.tile(v.reshape(1, -1), (8, 1))


def kernel(x, node_batch, edge_index, edge_attr, Wf, bf, Ws, bs, W1, b1,
           bn_w, bn_b, W2, b2, ln_w, ln_b):
  src = edge_index[0].astype(jnp.int32)
  dst = edge_index[1].astype(jnp.int32)
  nb = node_batch.astype(jnp.int32)
  onehot = (nb[:, None] == jnp.arange(_G, dtype=jnp.int32)[None, :])
  onehot = onehot.astype(jnp.float32)
  zeros = jnp.zeros((_N, _C), jnp.float32)
  dsta, dstb = dst[:_EH], dst[_EH:]
  srca, srcb = src[:_EH], src[_EH:]
  eaa, eab = edge_attr[:_EH], edge_attr[_EH:]

  wds = [jnp.concatenate([Wf[l, :_C], Ws[l, :_C]], axis=1) for l in range(_L)]
  wsrs = [jnp.concatenate([Wf[l, _C:2 * _C], Ws[l, _C:2 * _C]], axis=1)
          for l in range(_L)]

  td = ts = None
  for l in range(_L):
    wfe = Wf[l, 2 * _C:]
    wse = Ws[l, 2 * _C:]
    bf8, bs8 = _r8(bf[l]), _r8(bs[l])

    if l == 0:
      td, ts = _proj(x, wds[0], wsrs[0])
    gda, gsa = _sc_gather(td, ts, dsta, srca)
    gdb, gsb = _sc_gather(td, ts, dstb, srcb)
    msga = _edge(gda, gsa, eaa, wfe, wse, bf8, bs8)
    msgb = _edge(gdb, gsb, eab, wfe, wse, bf8, bs8)
    pa = _sc_scatter(msga, dsta, zeros)
    pb = _sc_scatter(msgb, dstb, zeros)
    x1, h, stats = _mlp1(x, pa, pb, W1[l], _r8(b1[l]))
    x2, seg = _mlp2(h, x1, stats, onehot, _r8(bn_w[l]), _r8(bn_b[l]),
                    W2[l], _r8(b2[l]))
    if l + 1 < _L:
      x, td, ts = _gln(x2, onehot, seg, _r8(ln_w[l]), _r8(ln_b[l]),
                       wds[l + 1], wsrs[l + 1])
    else:
      x = _gln(x2, onehot, seg, _r8(ln_w[l]), _r8(ln_b[l]))
  return x


# speedup vs baseline: 3.3730x; 1.0170x over previous
import functools

import jax
import jax.numpy as jnp
from jax import lax
from jax.experimental import pallas as pl
from jax.experimental.pallas import tpu as pltpu
from jax.experimental.pallas import tpu_sc as plsc

_L = 3
_C = 128
_D = 16
_H = 4 * _C
_N = 10000
_E = 320000
_G = 16
_EPS = 1e-5

_NC = 2
_NS = 16
_NW = _NC * _NS
_EH = _E // 2
_EW = _EH // _NW
_CH = 128
_NFULL = _EW // _CH
_NPAIR = _NFULL // 2
_CREM = 2 * _NPAIR
_TAIL = _EW - _NFULL * _CH
_TOFF = _NFULL * _CH

_TN = 2000
_TE = 4000


def _sc_mesh():
  return plsc.VectorSubcoreMesh(core_axis_name="c", subcore_axis_name="s")


def _sc_gather(td, ts, dst, src):
  @functools.partial(
      pl.kernel,
      mesh=_sc_mesh(),
      out_type=(
          jax.ShapeDtypeStruct((_EH, _C), jnp.int32),
          jax.ShapeDtypeStruct((_EH, _C), jnp.int32),
      ),
      scratch_types=[
          pltpu.VMEM((_EW,), jnp.int32),
          pltpu.VMEM((_EW,), jnp.int32),
          [pltpu.VMEM((_CH, _C), jnp.int32)] * 2,
          [pltpu.VMEM((_CH, _C), jnp.int32)] * 2,
          pltpu.VMEM((_TAIL, _C), jnp.int32),
          pltpu.VMEM((_TAIL, _C), jnp.int32),
          [pltpu.SemaphoreType.DMA] * 2,
          [pltpu.SemaphoreType.DMA] * 2,
          [pltpu.SemaphoreType.DMA] * 2,
          [pltpu.SemaphoreType.DMA] * 2,
          pltpu.SemaphoreType.DMA,
      ],
  )
  def k(td_hbm, ts_hbm, dst_hbm, src_hbm, gd_hbm, gs_hbm,
        idxd, idxs, rowsd, rowss, taild, tails,
        gsemd, gsems, wsemd, wsems, tsem):
    wid = lax.axis_index("s") * _NC + lax.axis_index("c")
    base = wid * _EW
    pltpu.sync_copy(dst_hbm.at[pl.ds(base, _EW)], idxd)
    pltpu.sync_copy(src_hbm.at[pl.ds(base, _EW)], idxs)

    def body(g, carry):
      gds = []
      for b in range(2):
        c = 2 * g + b
        osl = pl.ds(base + c * _CH, _CH)
        isl = pl.ds(c * _CH, _CH)

        @pl.when(g > 0)
        def _():
          pltpu.make_async_copy(rowsd[b], gd_hbm.at[osl], wsemd[b]).wait()
          pltpu.make_async_copy(rowss[b], gs_hbm.at[osl], wsems[b]).wait()

        gds.append((
            pltpu.async_copy(td_hbm.at[idxd.at[isl]], rowsd[b], gsemd[b]),
            pltpu.async_copy(ts_hbm.at[idxs.at[isl]], rowss[b], gsems[b]),
        ))
      for b in range(2):
        c = 2 * g + b
        osl = pl.ds(base + c * _CH, _CH)
        cpd, cps = gds[b]
        cpd.wait()
        cps.wait()
        pltpu.async_copy(rowsd[b], gd_hbm.at[osl], wsemd[b])
        pltpu.async_copy(rowss[b], gs_hbm.at[osl], wsems[b])
      return carry

    lax.fori_loop(0, _NPAIR, body, 0)

    osl = pl.ds(base + _CREM * _CH, _CH)
    pltpu.make_async_copy(rowsd[0], gd_hbm.at[osl], wsemd[0]).wait()
    pltpu.make_async_copy(rowss[0], gs_hbm.at[osl], wsems[0]).wait()
    isl = pl.ds(_CREM * _CH, _CH)
    cpd = pltpu.async_copy(td_hbm.at[idxd.at[isl]], rowsd[0], gsemd[0])
    cps = pltpu.async_copy(ts_hbm.at[idxs.at[isl]], rowss[0], gsems[0])
    cpd.wait()
    cps.wait()
    pltpu.sync_copy(rowsd[0], gd_hbm.at[osl])
    pltpu.sync_copy(rowss[0], gs_hbm.at[osl])

    tsl = pl.ds(base + _TOFF, _TAIL)
    cpd = pltpu.async_copy(td_hbm.at[idxd.at[pl.ds(_TOFF, _TAIL)]], taild, tsem)
    cpd.wait()
    cps = pltpu.async_copy(ts_hbm.at[idxs.at[pl.ds(_TOFF, _TAIL)]], tails, tsem)
    cps.wait()
    pltpu.sync_copy(taild, gd_hbm.at[tsl])
    pltpu.sync_copy(tails, gs_hbm.at[tsl])
    osl = pl.ds(base + (_CREM - 1) * _CH, _CH)
    pltpu.make_async_copy(rowsd[1], gd_hbm.at[osl], wsemd[1]).wait()
    pltpu.make_async_copy(rowss[1], gs_hbm.at[osl], wsems[1]).wait()

  return k(td, ts, dst, src)


def _sc_scatter(msg, dst, zeros):
  @functools.partial(
      pl.kernel,
      mesh=_sc_mesh(),
      out_type=jax.ShapeDtypeStruct((_NC, _N, _C), jnp.float32),
      scratch_types=[
          [pltpu.VMEM((_CH,), jnp.int32)] * 2,
          [pltpu.VMEM((_CH, _C), jnp.float32)] * 2,
          pltpu.VMEM((_TAIL,), jnp.int32),
          pltpu.VMEM((_TAIL, _C), jnp.float32),
          pltpu.VMEM_SHARED((_N, _C), jnp.float32),
          [pltpu.SemaphoreType.DMA] * 2,
          [pltpu.SemaphoreType.DMA] * 2,
          [pltpu.SemaphoreType.DMA] * 2,
          pltpu.SemaphoreType.DMA,
      ],
  )
  def k(msg_hbm, dst_hbm, z_hbm, out_hbm, idx, rows, idxt, rowst, acc,
        isem, lsem, ssem, tsem):
    cid = lax.axis_index("c")
    sid = lax.axis_index("s")
    wid = sid * _NC + cid

    @pl.when(sid == 0)
    def _():
      pltpu.sync_copy(z_hbm, acc)

    plsc.subcore_barrier()
    base = wid * _EW

    def body(g, carry):
      lds = []
      for b in range(2):
        c = 2 * g + b
        osl = pl.ds(base + c * _CH, _CH)

        @pl.when(g > 0)
        def _():
          pltpu.make_async_copy(rows[b], acc.at[idx[b]], ssem[b]).wait()

        lds.append((
            pltpu.async_copy(dst_hbm.at[osl], idx[b], isem[b]),
            pltpu.async_copy(msg_hbm.at[osl], rows[b], lsem[b]),
        ))
      for b in range(2):
        cpi, cpm = lds[b]
        cpi.wait()
        cpm.wait()
        pltpu.async_copy(rows[b], acc.at[idx[b]], ssem[b], add=True)
      return carry

    lax.fori_loop(0, _NPAIR, body, 0)

    pltpu.make_async_copy(rows[0], acc.at[idx[0]], ssem[0]).wait()
    osl = pl.ds(base + _CREM * _CH, _CH)
    cpi = pltpu.async_copy(dst_hbm.at[osl], idx[0], isem[0])
    cpm = pltpu.async_copy(msg_hbm.at[osl], rows[0], lsem[0])
    cpi.wait()
    cpm.wait()
    pltpu.sync_copy(rows[0], acc.at[idx[0]], add=True)

    pltpu.make_async_copy(rows[1], acc.at[idx[1]], ssem[1]).wait()
    tsl = pl.ds(base + _TOFF, _TAIL)
    cpi = pltpu.async_copy(dst_hbm.at[tsl], idxt, tsem)
    cpi.wait()
    cpm = pltpu.async_copy(msg_hbm.at[tsl], rowst, tsem)
    cpm.wait()
    pltpu.sync_copy(rowst, acc.at[idxt], add=True)

    plsc.subcore_barrier()

    @pl.when(sid == 0)
    def _():
      pltpu.sync_copy(acc, out_hbm.at[cid])

  return k(msg, dst, zeros)


def _rne_bf16_bits(v):
  bits = lax.bitcast_convert_type(v, jnp.int32)
  return (bits + 0x7FFF + ((bits >> 16) & 1)) >> 16


def _pack2(f, s):
  return (_rne_bf16_bits(s) << 16) | (_rne_bf16_bits(f) & 0xFFFF)


def _unpack_lo(w):
  return lax.bitcast_convert_type(w << 16, jnp.float32)


def _unpack_hi(w):
  return lax.bitcast_convert_type(w & jnp.int32(-65536), jnp.float32)


def _proj(x, wd, ws):
  def body(x_ref, wd_ref, ws_ref, td_ref, ts_ref):
    xv = x_ref[...]
    pd = jnp.dot(xv, wd_ref[...], preferred_element_type=jnp.float32, precision=lax.Precision.HIGHEST)
    ps = jnp.dot(xv, ws_ref[...], preferred_element_type=jnp.float32, precision=lax.Precision.HIGHEST)
    td_ref[...] = _pack2(pd[:, :_C], pd[:, _C:])
    ts_ref[...] = _pack2(ps[:, :_C], ps[:, _C:])

  return pl.pallas_call(
      body,
      grid=(_N // _TN,),
      in_specs=[
          pl.BlockSpec((_TN, _C), lambda i: (i, 0)),
          pl.BlockSpec((_C, 2 * _C), lambda i: (0, 0)),
          pl.BlockSpec((_C, 2 * _C), lambda i: (0, 0)),
      ],
      out_specs=[
          pl.BlockSpec((_TN, _C), lambda i: (i, 0)),
          pl.BlockSpec((_TN, _C), lambda i: (i, 0)),
      ],
      out_shape=[jax.ShapeDtypeStruct((_N, _C), jnp.int32)] * 2,
  )(x, wd, ws)


def _edge(gd, gs, ea, wfe, wse, bf8, bs8):
  def body(gd_ref, gs_ref, ea_ref, wfe_ref, wse_ref, bf_ref, bs_ref, o_ref):
    eav = ea_ref[...]
    ef = jnp.dot(eav, wfe_ref[...], preferred_element_type=jnp.float32, precision=lax.Precision.HIGHEST)
    es = jnp.dot(eav, wse_ref[...], preferred_element_type=jnp.float32, precision=lax.Precision.HIGHEST)
    gd = gd_ref[...]
    gs = gs_ref[...]
    a = _unpack_lo(gd) + _unpack_lo(gs) + ef + bf_ref[0:1, :]
    b = _unpack_hi(gd) + _unpack_hi(gs) + es + bs_ref[0:1, :]
    gate = 0.5 + 0.5 * jnp.tanh(0.5 * a)
    core = jnp.maximum(b, 0.0) + jnp.log1p(jnp.exp(-jnp.abs(b)))
    o_ref[...] = gate * core

  ne = gd.shape[0]
  return pl.pallas_call(
      body,
      grid=(ne // _TE,),
      in_specs=[
          pl.BlockSpec((_TE, _C), lambda i: (i, 0)),
          pl.BlockSpec((_TE, _C), lambda i: (i, 0)),
          pl.BlockSpec((_TE, _D), lambda i: (i, 0)),
          pl.BlockSpec((_D, _C), lambda i: (0, 0)),
          pl.BlockSpec((_D, _C), lambda i: (0, 0)),
          pl.BlockSpec((8, _C), lambda i: (0, 0)),
          pl.BlockSpec((8, _C), lambda i: (0, 0)),
      ],
      out_specs=pl.BlockSpec((_TE, _C), lambda i: (i, 0)),
      out_shape=jax.ShapeDtypeStruct((ne, _C), jnp.float32),
  )(gd, gs, ea, wfe, wse, bf8, bs8)


def _mlp1(x, pa, pb, w1, b18):
  def body(x_ref, pa_ref, pb_ref, w1_ref, b1_ref, x1_ref, h_ref, s_ref):
    i = pl.program_id(0)
    x1 = (x_ref[...] + (pa_ref[0] + pa_ref[1]) + (pb_ref[0] + pb_ref[1]))
    x1_ref[...] = x1
    h = jnp.dot(x1, w1_ref[...], preferred_element_type=jnp.float32, precision=lax.Precision.HIGHEST)
    h = h + b1_ref[0:1, :]
    h_ref[...] = h
    upd = jnp.concatenate(
        [
            jnp.sum(h, axis=0, keepdims=True),
            jnp.sum(h * h, axis=0, keepdims=True),
            jnp.zeros((6, _H), jnp.float32),
        ],
        axis=0,
    )

    @pl.when(i == 0)
    def _():
      s_ref[...] = upd

    @pl.when(i > 0)
    def _():
      s_ref[...] += upd

  return pl.pallas_call(
      body,
      grid=(_N // _TN,),
      in_specs=[
          pl.BlockSpec((_TN, _C), lambda i: (i, 0)),
          pl.BlockSpec((2, _TN, _C), lambda i: (0, i, 0)),
          pl.BlockSpec((2, _TN, _C), lambda i: (0, i, 0)),
          pl.BlockSpec((_C, _H), lambda i: (0, 0)),
          pl.BlockSpec((8, _H), lambda i: (0, 0)),
      ],
      out_specs=[
          pl.BlockSpec((_TN, _C), lambda i: (i, 0)),
          pl.BlockSpec((_TN, _H), lambda i: (i, 0)),
          pl.BlockSpec((8, _H), lambda i: (0, 0)),
      ],
      out_shape=[
          jax.ShapeDtypeStruct((_N, _C), jnp.float32),
          jax.ShapeDtypeStruct((_N, _H), jnp.float32),
          jax.ShapeDtypeStruct((8, _H), jnp.float32),
      ],
  )(x, pa, pb, w1, b18)


def _mlp2(h, x1, stats, onehot, bnw8, bnb8, w2, b28):
  def body(h_ref, x1_ref, s_ref, oh_ref, bnw_ref, bnb_ref, w2_ref, b2_ref,
           x2_ref, seg_ref):
    i = pl.program_id(0)
    mu = s_ref[0:1, :] / _N
    var = s_ref[1:2, :] / _N - mu * mu
    hn = (h_ref[...] - mu) * lax.rsqrt(var + _EPS)
    hn = hn * bnw_ref[0:1, :] + bnb_ref[0:1, :]
    hr = jnp.maximum(hn, 0.0)
    xp = jnp.dot(hr, w2_ref[...], preferred_element_type=jnp.float32, precision=lax.Precision.HIGHEST)
    x2 = x1_ref[...] + xp + b2_ref[0:1, :]
    x2_ref[...] = x2
    oh = oh_ref[...]
    dn = (((0,), (0,)), ((), ()))
    s1 = lax.dot_general(oh, x2, dn, preferred_element_type=jnp.float32, precision=lax.Precision.HIGHEST)
    s2 = lax.dot_general(oh, x2 * x2, dn, preferred_element_type=jnp.float32, precision=lax.Precision.HIGHEST)
    dg = lax.dot_general(oh, jnp.ones_like(x2), dn,
                         preferred_element_type=jnp.float32, precision=lax.Precision.HIGHEST)
    upd = jnp.concatenate([s1, s2, dg], axis=0)

    @pl.when(i == 0)
    def _():
      seg_ref[...] = upd

    @pl.when(i > 0)
    def _():
      seg_ref[...] += upd

  return pl.pallas_call(
      body,
      grid=(_N // _TN,),
      in_specs=[
          pl.BlockSpec((_TN, _H), lambda i: (i, 0)),
          pl.BlockSpec((_TN, _C), lambda i: (i, 0)),
          pl.BlockSpec((8, _H), lambda i: (0, 0)),
          pl.BlockSpec((_TN, _G), lambda i: (i, 0)),
          pl.BlockSpec((8, _H), lambda i: (0, 0)),
          pl.BlockSpec((8, _H), lambda i: (0, 0)),
          pl.BlockSpec((_H, _C), lambda i: (0, 0)),
          pl.BlockSpec((8, _C), lambda i: (0, 0)),
      ],
      out_specs=[
          pl.BlockSpec((_TN, _C), lambda i: (i, 0)),
          pl.BlockSpec((3 * _G, _C), lambda i: (0, 0)),
      ],
      out_shape=[
          jax.ShapeDtypeStruct((_N, _C), jnp.float32),
          jax.ShapeDtypeStruct((3 * _G, _C), jnp.float32),
      ],
  )(h, x1, stats, onehot, bnw8, bnb8, w2, b28)


def _gln(x2, onehot, seg, lnw8, lnb8, wd=None, ws=None):
  fuse = wd is not None

  def body(x2_ref, oh_ref, seg_ref, lnw_ref, lnb_ref, *rest):
    s1 = seg_ref[0:_G, :]
    s2 = seg_ref[_G:2 * _G, :]
    deg = seg_ref[2 * _G:3 * _G, 0:1]
    norm = jnp.maximum(deg, 1.0) * _C
    mean_g = jnp.sum(s1, axis=1, keepdims=True) / norm
    var_g = jnp.sum(s2, axis=1, keepdims=True) / norm - mean_g * mean_g
    inv_g = lax.rsqrt(var_g + _EPS)
    mean_b = jnp.broadcast_to(mean_g, (_G, _C))
    inv_b = jnp.broadcast_to(inv_g, (_G, _C))
    oh = oh_ref[...]
    m = jnp.dot(oh, mean_b, preferred_element_type=jnp.float32, precision=lax.Precision.HIGHEST)
    iv = jnp.dot(oh, inv_b, preferred_element_type=jnp.float32, precision=lax.Precision.HIGHEST)
    xo = (x2_ref[...] - m) * iv * lnw_ref[0:1, :] + lnb_ref[0:1, :]
    if fuse:
      wd_ref, ws_ref, o_ref, td_ref, ts_ref = rest
      o_ref[...] = xo
      pd = jnp.dot(xo, wd_ref[...], preferred_element_type=jnp.float32, precision=lax.Precision.HIGHEST)
      ps = jnp.dot(xo, ws_ref[...], preferred_element_type=jnp.float32, precision=lax.Precision.HIGHEST)
      td_ref[...] = _pack2(pd[:, :_C], pd[:, _C:])
      ts_ref[...] = _pack2(ps[:, :_C], ps[:, _C:])
    else:
      rest[0][...] = xo

  in_specs = [
      pl.BlockSpec((_TN, _C), lambda i: (i, 0)),
      pl.BlockSpec((_TN, _G), lambda i: (i, 0)),
      pl.BlockSpec((3 * _G, _C), lambda i: (0, 0)),
      pl.BlockSpec((8, _C), lambda i: (0, 0)),
      pl.BlockSpec((8, _C), lambda i: (0, 0)),
  ]
  args = [x2, onehot, seg, lnw8, lnb8]
  out_specs = [pl.BlockSpec((_TN, _C), lambda i: (i, 0))]
  out_shape = [jax.ShapeDtypeStruct((_N, _C), jnp.float32)]
  if fuse:
    in_specs += [pl.BlockSpec((_C, 2 * _C), lambda i: (0, 0))] * 2
    args += [wd, ws]
    out_specs += [pl.BlockSpec((_TN, _C), lambda i: (i, 0))] * 2
    out_shape += [jax.ShapeDtypeStruct((_N, _C), jnp.int32)] * 2
  res = pl.pallas_call(
      body,
      grid=(_N // _TN,),
      in_specs=in_specs,
      out_specs=out_specs,
      out_shape=out_shape,
  )(*args)
  return res if fuse else res[0]


def _r8(v):
  return jnp.tile(v.reshape(1, -1), (8, 1))


def kernel(x, node_batch, edge_index, edge_attr, Wf, bf, Ws, bs, W1, b1,
           bn_w, bn_b, W2, b2, ln_w, ln_b):
  src = edge_index[0].astype(jnp.int32)
  dst = edge_index[1].astype(jnp.int32)
  nb = node_batch.astype(jnp.int32)
  onehot = (nb[:, None] == jnp.arange(_G, dtype=jnp.int32)[None, :])
  onehot = onehot.astype(jnp.float32)
  zeros = jnp.zeros((_N, _C), jnp.float32)
  dsta, dstb = dst[:_EH], dst[_EH:]
  srca, srcb = src[:_EH], src[_EH:]
  eaa, eab = edge_attr[:_EH], edge_attr[_EH:]

  wds = [jnp.concatenate([Wf[l, :_C], Ws[l, :_C]], axis=1) for l in range(_L)]
  wsrs = [jnp.concatenate([Wf[l, _C:2 * _C], Ws[l, _C:2 * _C]], axis=1)
          for l in range(_L)]

  td = ts = None
  for l in range(_L):
    wfe = Wf[l, 2 * _C:]
    wse = Ws[l, 2 * _C:]
    bf8, bs8 = _r8(bf[l]), _r8(bs[l])

    if l == 0:
      td, ts = _proj(x, wds[0], wsrs[0])
    gda, gsa = _sc_gather(td, ts, dsta, srca)
    gdb, gsb = _sc_gather(td, ts, dstb, srcb)
    msga = _edge(gda, gsa, eaa, wfe, wse, bf8, bs8)
    msgb = _edge(gdb, gsb, eab, wfe, wse, bf8, bs8)
    pa = _sc_scatter(msga, dsta, zeros)
    pb = _sc_scatter(msgb, dstb, zeros)
    x1, h, stats = _mlp1(x, pa, pb, W1[l], _r8(b1[l]))
    x2, seg = _mlp2(h, x1, stats, onehot, _r8(bn_w[l]), _r8(bn_b[l]),
                    W2[l], _r8(b2[l]))
    if l + 1 < _L:
      x, td, ts = _gln(x2, onehot, seg, _r8(ln_w[l]), _r8(ln_b[l]),
                       wds[l + 1], wsrs[l + 1])
    else:
      x = _gln(x2, onehot, seg, _r8(ln_w[l]), _r8(ln_b[l]))
  return x
